# Initial kernel scaffold; baseline (speedup 1.0000x reference)
import math

import jax
import jax.numpy as jnp
import numpy as np
from jax.experimental import pallas as pl

B, T, NUM_TYPES, K, H = 4, 256, 128, 128, 32
RADIUS, MAX_NBR, MAX_REP = 5.0, 32, 2
C = (2 * MAX_REP + 1) * (2 * MAX_REP + 1)
NCAND = T * C
_SQRT_2PI = math.sqrt(2.0 * math.pi)

_RG = 8
_BI = 32
_E = _BI * MAX_NBR


def _select_body(p2x_ref, p2y_ref, p2z_ref, px_ref, py_ref, pz_ref,
                 idx_ref, d2_ref):
    p2x = p2x_ref[0, 0, :]
    p2y = p2y_ref[0, 0, :]
    p2z = p2z_ref[0, 0, :]
    px = px_ref[0]
    py = py_ref[0]
    pz = pz_ref[0]
    dx = p2x - px
    dy = p2y - py
    dz = p2z - pz
    d2 = dx * dx + dy * dy + dz * dz
    ok = (d2 <= RADIUS * RADIUS) & (d2 > 1e-4)
    d2m = jnp.where(ok, d2, jnp.inf)
    iota = jax.lax.broadcasted_iota(jnp.int32, (_RG, NCAND), 1)
    idx_cols = []
    d2_cols = []
    for _ in range(MAX_NBR):
        rowmin = jnp.min(d2m, axis=1, keepdims=True)
        cand = jnp.where(d2m == rowmin, iota, jnp.int32(2 ** 30))
        amin = jnp.min(cand, axis=1, keepdims=True)
        idx_cols.append(amin)
        d2_cols.append(rowmin)
        d2m = jnp.where(iota == amin, jnp.inf, d2m)
    idx_ref[0] = jnp.concatenate(idx_cols, axis=1)
    d2_ref[0] = jnp.concatenate(d2_cols, axis=1)


def _edge_body(idx_ref, d2_ref, ai_ref, atb_ref, mulm_ref, biasm_ref,
               means_ref, stds_ref, w1_ref, b1_ref, w2_ref, b2_ref, out_ref):
    k = idx_ref[...]
    d2s = d2_ref[...]
    valid = d2s <= RADIUS * RADIUS
    j = (k // C).astype(jnp.float32).reshape(_E, 1)

    iota_t = jax.lax.broadcasted_iota(jnp.int32, (_E, T), 1).astype(jnp.float32)
    oj = (j == iota_t).astype(jnp.float32)
    atb = atb_ref[0, 0, :]
    a_j = jnp.sum(oj * atb, axis=1, keepdims=True)

    a_i = ai_ref[...]
    iota_k = jax.lax.broadcasted_iota(jnp.int32, (_BI, NUM_TYPES), 1).astype(jnp.float32)
    ohi = (a_i == iota_k).astype(jnp.float32)
    rows_mul = jnp.dot(ohi, mulm_ref[...], preferred_element_type=jnp.float32)
    rows_bias = jnp.dot(ohi, biasm_ref[...], preferred_element_type=jnp.float32)
    rows_mul_e = jnp.broadcast_to(
        rows_mul[:, None, :], (_BI, MAX_NBR, NUM_TYPES)).reshape(_E, NUM_TYPES)
    rows_bias_e = jnp.broadcast_to(
        rows_bias[:, None, :], (_BI, MAX_NBR, NUM_TYPES)).reshape(_E, NUM_TYPES)
    iota_e = jax.lax.broadcasted_iota(jnp.int32, (_E, NUM_TYPES), 1).astype(jnp.float32)
    ohj = (a_j == iota_e).astype(jnp.float32)
    mul_e = jnp.sum(ohj * rows_mul_e, axis=1, keepdims=True)
    bias_e = jnp.sum(ohj * rows_bias_e, axis=1, keepdims=True)

    d2c = jnp.where(valid, d2s, 0.0).reshape(_E, 1)
    dist = jnp.sqrt(d2c + 1e-12)
    x = mul_e * dist + bias_e

    std = jnp.abs(stds_ref[0, :]) + 1e-5
    pre = (x - means_ref[0, :][None, :]) / std[None, :]
    g = jnp.exp(-0.5 * pre * pre) / (_SQRT_2PI * std[None, :])
    h1 = jax.nn.gelu(jnp.dot(g, w1_ref[...], preferred_element_type=jnp.float32)
                     + b1_ref[0, :][None, :])
    h = jnp.dot(h1, w2_ref[...], preferred_element_type=jnp.float32) + b2_ref[0, :][None, :]
    h = jnp.where(valid.reshape(_E, 1), h, 0.0)

    for i in range(_BI):
        hi = h[i * MAX_NBR:(i + 1) * MAX_NBR, :]
        si = oj[i * MAX_NBR:(i + 1) * MAX_NBR, :]
        out_ref[0, :, i, :] = jax.lax.dot_general(
            hi, si, (((0,), (0,)), ((), ())), preferred_element_type=jnp.float32)


def kernel(pos, atom_types, natoms, cell, means, stds, mul_w, bias_w,
           W1, b1, W2, b2):
    del natoms
    cross_a2a3 = jnp.cross(cell[:, 1], cell[:, 2])
    cell_vol = jnp.sum(cell[:, 0] * cross_a2a3, axis=-1, keepdims=True)
    crosses = [cross_a2a3, jnp.cross(cell[:, 2], cell[:, 0]),
               jnp.cross(cell[:, 0], cell[:, 1])]
    pbc = (True, True, False)
    rep_caps = []
    for dim in range(3):
        if pbc[dim]:
            inv = jnp.linalg.norm(crosses[dim] / cell_vol, axis=-1)
            rep_caps.append(jnp.minimum(jnp.max(jnp.ceil(RADIUS * inv)), float(MAX_REP)))
        else:
            rep_caps.append(jnp.asarray(0.0, jnp.float32))
    static_reps = [MAX_REP if pbc[dim] else 0 for dim in range(3)]
    cells_per_dim = [np.arange(-r, r + 1, dtype=np.float32) for r in static_reps]
    mg = np.meshgrid(*cells_per_dim, indexing="ij")
    unit_cell = jnp.asarray(np.stack([m.reshape(-1) for m in mg], axis=1))
    cell_ok = jnp.ones((C,), bool)
    for dim in range(3):
        cell_ok = cell_ok & (jnp.abs(unit_cell[:, dim]) <= rep_caps[dim])
    pbc_off = jnp.einsum("ci,bij->bcj", unit_cell, cell)
    pos2 = pos[:, :, None, :] + pbc_off[:, None, :, :]
    pos2 = jnp.where(cell_ok[None, None, :, None], pos2, 1e9)
    pos2 = pos2.reshape(B, 1, NCAND, 3)

    p2x = pos2[..., 0]
    p2y = pos2[..., 1]
    p2z = pos2[..., 2]
    pos_g = pos.reshape(B * T // _RG, _RG, 3)
    px = pos_g[..., 0:1]
    py = pos_g[..., 1:2]
    pz = pos_g[..., 2:3]

    ng = B * T // _RG
    sel_idx, sel_d2 = pl.pallas_call(
        _select_body,
        grid=(ng,),
        in_specs=[
            pl.BlockSpec((1, 1, NCAND), lambda g: (g // (T // _RG), 0, 0)),
            pl.BlockSpec((1, 1, NCAND), lambda g: (g // (T // _RG), 0, 0)),
            pl.BlockSpec((1, 1, NCAND), lambda g: (g // (T // _RG), 0, 0)),
            pl.BlockSpec((1, _RG, 1), lambda g: (g, 0, 0)),
            pl.BlockSpec((1, _RG, 1), lambda g: (g, 0, 0)),
            pl.BlockSpec((1, _RG, 1), lambda g: (g, 0, 0)),
        ],
        out_specs=[
            pl.BlockSpec((1, _RG, MAX_NBR), lambda g: (g, 0, 0)),
            pl.BlockSpec((1, _RG, MAX_NBR), lambda g: (g, 0, 0)),
        ],
        out_shape=[
            jax.ShapeDtypeStruct((ng, _RG, MAX_NBR), jnp.int32),
            jax.ShapeDtypeStruct((ng, _RG, MAX_NBR), jnp.float32),
        ],
    )(p2x, p2y, p2z, px, py, pz)
    sel_idx = sel_idx.reshape(B * T, MAX_NBR)
    sel_d2 = sel_d2.reshape(B * T, MAX_NBR)

    atype_f = atom_types.astype(jnp.float32)
    ai = atype_f.reshape(B * T, 1)
    atb = atype_f.reshape(B, 1, T)
    mulm = mul_w.reshape(NUM_TYPES, NUM_TYPES)
    biasm = bias_w.reshape(NUM_TYPES, NUM_TYPES)

    nsteps = B * T // _BI
    steps_per_b = T // _BI
    out = pl.pallas_call(
        _edge_body,
        grid=(nsteps,),
        in_specs=[
            pl.BlockSpec((_BI, MAX_NBR), lambda g: (g, 0)),
            pl.BlockSpec((_BI, MAX_NBR), lambda g: (g, 0)),
            pl.BlockSpec((_BI, 1), lambda g: (g, 0)),
            pl.BlockSpec((1, 1, T), lambda g: (g // steps_per_b, 0, 0)),
            pl.BlockSpec((NUM_TYPES, NUM_TYPES), lambda g: (0, 0)),
            pl.BlockSpec((NUM_TYPES, NUM_TYPES), lambda g: (0, 0)),
            pl.BlockSpec((1, K), lambda g: (0, 0)),
            pl.BlockSpec((1, K), lambda g: (0, 0)),
            pl.BlockSpec((K, K), lambda g: (0, 0)),
            pl.BlockSpec((1, K), lambda g: (0, 0)),
            pl.BlockSpec((K, H), lambda g: (0, 0)),
            pl.BlockSpec((1, H), lambda g: (0, 0)),
        ],
        out_specs=pl.BlockSpec(
            (1, H, _BI, T), lambda g: (g // steps_per_b, 0, g % steps_per_b, 0)),
        out_shape=jax.ShapeDtypeStruct((B, H, T, T), jnp.float32),
    )(sel_idx, sel_d2, ai, atb, mulm, biasm,
      means.reshape(1, K), stds.reshape(1, K), W1, b1.reshape(1, K),
      W2, b2.reshape(1, H))
    return out


# speedup vs baseline: 3.3106x; 3.3106x over previous
import math

import jax
import jax.numpy as jnp
import numpy as np
from jax.experimental import pallas as pl

B, T, NUM_TYPES, K, H = 4, 256, 128, 128, 32
RADIUS, MAX_NBR, MAX_REP = 5.0, 32, 2
C = (2 * MAX_REP + 1) * (2 * MAX_REP + 1)
NCAND = T * C
_SQRT_2PI = math.sqrt(2.0 * math.pi)

_RG = 8
_BI = 32
_E = _BI * MAX_NBR


def _select_body(p2x_ref, p2y_ref, p2z_ref, px_ref, py_ref, pz_ref,
                 idx_ref, d2_ref):
    p2x = p2x_ref[0, 0, :]
    p2y = p2y_ref[0, 0, :]
    p2z = p2z_ref[0, 0, :]
    px = px_ref[0]
    py = py_ref[0]
    pz = pz_ref[0]
    dx = p2x - px
    dy = p2y - py
    dz = p2z - pz
    d2 = dx * dx + dy * dy + dz * dz
    ok = (d2 <= RADIUS * RADIUS) & (d2 > 1e-4)
    d2m = jnp.where(ok, d2, jnp.inf)
    iota = jax.lax.broadcasted_iota(jnp.int32, (_RG, NCAND), 1)
    idx_cols = []
    d2_cols = []
    for _ in range(MAX_NBR):
        rowmin = jnp.min(d2m, axis=1, keepdims=True)
        cand = jnp.where(d2m == rowmin, iota, jnp.int32(2 ** 30))
        amin = jnp.min(cand, axis=1, keepdims=True)
        idx_cols.append(amin)
        d2_cols.append(rowmin)
        d2m = jnp.where(iota == amin, jnp.inf, d2m)
    idx_ref[0] = jnp.concatenate(idx_cols, axis=1)
    d2_ref[0] = jnp.concatenate(d2_cols, axis=1)


def _edge_body(idx_ref, d2_ref, ai_ref, atb_ref, mulm_ref, biasm_ref,
               means_ref, stds_ref, w1_ref, b1_ref, w2_ref, b2_ref, out_ref):
    k = idx_ref[0]
    d2s = d2_ref[0]
    valid = d2s <= RADIUS * RADIUS
    j = (k // C).astype(jnp.float32)

    iota_t = jax.lax.broadcasted_iota(jnp.int32, (_E, T), 1).astype(jnp.float32)
    oj = (j == iota_t).astype(jnp.float32)
    atb = atb_ref[0]
    a_j = jnp.sum(oj * atb, axis=1, keepdims=True)

    a_i = ai_ref[0]
    iota_k = jax.lax.broadcasted_iota(jnp.int32, (_E, NUM_TYPES), 1).astype(jnp.float32)
    ohi = (a_i == iota_k).astype(jnp.float32)
    rows_mul = jnp.dot(ohi, mulm_ref[...], preferred_element_type=jnp.float32, precision=jax.lax.Precision.HIGHEST)
    rows_bias = jnp.dot(ohi, biasm_ref[...], preferred_element_type=jnp.float32, precision=jax.lax.Precision.HIGHEST)
    ohj = (a_j == iota_k).astype(jnp.float32)
    mul_e = jnp.sum(ohj * rows_mul, axis=1, keepdims=True)
    bias_e = jnp.sum(ohj * rows_bias, axis=1, keepdims=True)

    d2c = jnp.where(valid, d2s, 0.0)
    dist = jnp.sqrt(d2c + 1e-12)
    x = mul_e * dist + bias_e

    std = jnp.abs(stds_ref[...]) + 1e-5
    pre = (x - means_ref[...]) / std
    g = jnp.exp(-0.5 * pre * pre) / (_SQRT_2PI * std)
    h1 = jax.nn.gelu(jnp.dot(g, w1_ref[...], preferred_element_type=jnp.float32, precision=jax.lax.Precision.HIGHEST)
                     + b1_ref[...])
    h = jnp.dot(h1, w2_ref[...], preferred_element_type=jnp.float32, precision=jax.lax.Precision.HIGHEST) + b2_ref[...]
    h = jnp.where(valid, h, 0.0)

    for i in range(_BI):
        hi = h[i * MAX_NBR:(i + 1) * MAX_NBR, :]
        si = oj[i * MAX_NBR:(i + 1) * MAX_NBR, :]
        out_ref[0, :, i, :] = jax.lax.dot_general(
            hi, si, (((0,), (0,)), ((), ())), preferred_element_type=jnp.float32, precision=jax.lax.Precision.HIGHEST)


def kernel(pos, atom_types, natoms, cell, means, stds, mul_w, bias_w,
           W1, b1, W2, b2):
    del natoms
    cross_a2a3 = jnp.cross(cell[:, 1], cell[:, 2])
    cell_vol = jnp.sum(cell[:, 0] * cross_a2a3, axis=-1, keepdims=True)
    crosses = [cross_a2a3, jnp.cross(cell[:, 2], cell[:, 0]),
               jnp.cross(cell[:, 0], cell[:, 1])]
    pbc = (True, True, False)
    rep_caps = []
    for dim in range(3):
        if pbc[dim]:
            inv = jnp.linalg.norm(crosses[dim] / cell_vol, axis=-1)
            rep_caps.append(jnp.minimum(jnp.max(jnp.ceil(RADIUS * inv)), float(MAX_REP)))
        else:
            rep_caps.append(jnp.asarray(0.0, jnp.float32))
    static_reps = [MAX_REP if pbc[dim] else 0 for dim in range(3)]
    cells_per_dim = [np.arange(-r, r + 1, dtype=np.float32) for r in static_reps]
    mg = np.meshgrid(*cells_per_dim, indexing="ij")
    unit_cell = jnp.asarray(np.stack([m.reshape(-1) for m in mg], axis=1))
    cell_ok = jnp.ones((C,), bool)
    for dim in range(3):
        cell_ok = cell_ok & (jnp.abs(unit_cell[:, dim]) <= rep_caps[dim])
    pbc_off = jnp.einsum("ci,bij->bcj", unit_cell, cell)
    pos2 = pos[:, :, None, :] + pbc_off[:, None, :, :]
    pos2 = jnp.where(cell_ok[None, None, :, None], pos2, 1e9)
    pos2 = pos2.reshape(B, 1, NCAND, 3)

    p2x = pos2[..., 0]
    p2y = pos2[..., 1]
    p2z = pos2[..., 2]
    pos_g = pos.reshape(B * T // _RG, _RG, 3)
    px = pos_g[..., 0:1]
    py = pos_g[..., 1:2]
    pz = pos_g[..., 2:3]

    ng = B * T // _RG
    sel_idx, sel_d2 = pl.pallas_call(
        _select_body,
        grid=(ng,),
        in_specs=[
            pl.BlockSpec((1, 1, NCAND), lambda g: (g // (T // _RG), 0, 0)),
            pl.BlockSpec((1, 1, NCAND), lambda g: (g // (T // _RG), 0, 0)),
            pl.BlockSpec((1, 1, NCAND), lambda g: (g // (T // _RG), 0, 0)),
            pl.BlockSpec((1, _RG, 1), lambda g: (g, 0, 0)),
            pl.BlockSpec((1, _RG, 1), lambda g: (g, 0, 0)),
            pl.BlockSpec((1, _RG, 1), lambda g: (g, 0, 0)),
        ],
        out_specs=[
            pl.BlockSpec((1, _RG, MAX_NBR), lambda g: (g, 0, 0)),
            pl.BlockSpec((1, _RG, MAX_NBR), lambda g: (g, 0, 0)),
        ],
        out_shape=[
            jax.ShapeDtypeStruct((ng, _RG, MAX_NBR), jnp.int32),
            jax.ShapeDtypeStruct((ng, _RG, MAX_NBR), jnp.float32),
        ],
    )(p2x, p2y, p2z, px, py, pz)
    nsteps = B * T // _BI
    steps_per_b = T // _BI
    sel_idx = sel_idx.reshape(nsteps, _E, 1)
    sel_d2 = sel_d2.reshape(nsteps, _E, 1)

    atype_f = atom_types.astype(jnp.float32)
    ai = jnp.broadcast_to(atype_f.reshape(B * T, 1), (B * T, MAX_NBR))
    ai = ai.reshape(nsteps, _E, 1)
    atb = atype_f.reshape(B, 1, T)
    mulm = mul_w.reshape(NUM_TYPES, NUM_TYPES)
    biasm = bias_w.reshape(NUM_TYPES, NUM_TYPES)

    out = pl.pallas_call(
        _edge_body,
        grid=(nsteps,),
        in_specs=[
            pl.BlockSpec((1, _E, 1), lambda g: (g, 0, 0)),
            pl.BlockSpec((1, _E, 1), lambda g: (g, 0, 0)),
            pl.BlockSpec((1, _E, 1), lambda g: (g, 0, 0)),
            pl.BlockSpec((1, 1, T), lambda g: (g // steps_per_b, 0, 0)),
            pl.BlockSpec((NUM_TYPES, NUM_TYPES), lambda g: (0, 0)),
            pl.BlockSpec((NUM_TYPES, NUM_TYPES), lambda g: (0, 0)),
            pl.BlockSpec((1, K), lambda g: (0, 0)),
            pl.BlockSpec((1, K), lambda g: (0, 0)),
            pl.BlockSpec((K, K), lambda g: (0, 0)),
            pl.BlockSpec((1, K), lambda g: (0, 0)),
            pl.BlockSpec((K, H), lambda g: (0, 0)),
            pl.BlockSpec((1, H), lambda g: (0, 0)),
        ],
        out_specs=pl.BlockSpec(
            (1, H, _BI, T), lambda g: (g // steps_per_b, 0, g % steps_per_b, 0)),
        out_shape=jax.ShapeDtypeStruct((B, H, T, T), jnp.float32),
    )(sel_idx, sel_d2, ai, atb, mulm, biasm,
      means.reshape(1, K), stds.reshape(1, K), W1, b1.reshape(1, K),
      W2, b2.reshape(1, H))
    return out


# speedup vs baseline: 8.7009x; 2.6282x over previous
import functools
import math

import jax
import jax.numpy as jnp
import numpy as np
from jax import lax
from jax.experimental import pallas as pl
from jax.experimental.pallas import tpu as pltpu
from jax.experimental.pallas import tpu_sc as plsc

B, T, NUM_TYPES, K, H = 4, 256, 128, 128, 32
RADIUS, MAX_NBR, MAX_REP = 5.0, 32, 2
C = (2 * MAX_REP + 1) * (2 * MAX_REP + 1)
NCAND = T * C
_SQRT_2PI = math.sqrt(2.0 * math.pi)

_RG = 8
_BI = 32
_E = _BI * MAX_NBR


_L = 16
_NW = 32
_RPW = (B * T) // _NW
_NCH = NCAND // _L
_R2 = RADIUS * RADIUS


def _sc_select_body(p2x_hbm, p2y_hbm, p2z_hbm, pxs_hbm, pys_hbm, pzs_hbm,
                    outk_hbm, outd_hbm,
                    p2x, p2y, p2z, pxs, pys, pzs,
                    cd2, ck, cmin, outk_v, outd_v):
    cid = lax.axis_index("c")
    sid = lax.axis_index("s")
    wid = sid * 2 + cid
    base = wid * _RPW
    b = base // T

    pltpu.sync_copy(p2x_hbm.at[b], p2x)
    pltpu.sync_copy(p2y_hbm.at[b], p2y)
    pltpu.sync_copy(p2z_hbm.at[b], p2z)
    pltpu.sync_copy(pxs_hbm.at[pl.ds(base, _RPW)], pxs)
    pltpu.sync_copy(pys_hbm.at[pl.ds(base, _RPW)], pys)
    pltpu.sync_copy(pzs_hbm.at[pl.ds(base, _RPW)], pzs)

    iota = lax.broadcasted_iota(jnp.int32, (_L,), 0)
    inf16 = jnp.full((_L,), jnp.inf, jnp.float32)

    def row_body(r, _):
        px = pxs[r]
        py = pys[r]
        pz = pzs[r]

        def ch_body(ci, cnt):
            off = ci * _L
            dx = p2x[pl.ds(off, _L)] - px
            dy = p2y[pl.ds(off, _L)] - py
            dz = p2z[pl.ds(off, _L)] - pz
            d2 = dx * dx + dy * dy + dz * dz
            m = (d2 <= _R2) & (d2 > 1e-4)
            plsc.store_compressed(cd2.at[pl.ds(cnt, _L)], d2, mask=m)
            plsc.store_compressed(ck.at[pl.ds(cnt, _L)], off + iota, mask=m)
            return cnt + jnp.sum(m.astype(jnp.int32))

        cnt = lax.fori_loop(0, _NCH, ch_body, jnp.int32(0), unroll=4)
        cd2[pl.ds(cnt, _L)] = inf16

        nch_c = (cnt + _L - 1) // _L
        lane0 = iota == 0

        def cm_body(ci, _):
            mv = jnp.min(cd2[pl.ds(ci * _L, _L)])
            plsc.store_compressed(cmin.at[pl.ds(ci, _L)],
                                  jnp.full((_L,), mv, jnp.float32), mask=lane0)
            return 0

        lax.fori_loop(0, nch_c, cm_body, 0)
        cmin[pl.ds(nch_c, _L)] = inf16
        ncm = (nch_c + _L - 1) // _L

        def ex_body(n, carry):
            kacc, dacc = carry

            def gm_body(ci, gcarry):
                bv, bi = gcarry
                v = cmin[pl.ds(ci * _L, _L)]
                mv = jnp.min(v)
                lane = jnp.min(plsc.all_reduce_ffs(v == mv))
                better = mv < bv
                return (jnp.where(better, mv, bv),
                        jnp.where(better, ci * _L + lane, bi))

            gmin, fch = lax.fori_loop(0, ncm, gm_body,
                                      (jnp.float32(jnp.inf), jnp.int32(0)))
            coff = fch * _L
            v = cd2[pl.ds(coff, _L)]
            lane = jnp.min(plsc.all_reduce_ffs(v == gmin))
            vk = ck[pl.ds(coff, _L)]
            kval = jnp.min(jnp.where(iota == lane, vk, jnp.int32(2 ** 30)))
            kacc = jnp.where(iota == (n % _L), jnp.full((_L,), kval, jnp.int32), kacc)
            dacc = jnp.where(iota == (n % _L), jnp.full((_L,), gmin, jnp.float32), dacc)
            v2 = jnp.where(iota == lane, jnp.inf, v)
            cd2[pl.ds(coff, _L)] = v2
            plsc.store_compressed(cmin.at[pl.ds(fch, _L)],
                                  jnp.full((_L,), jnp.min(v2), jnp.float32),
                                  mask=lane0)
            return kacc, dacc

        z16i = jnp.zeros((_L,), jnp.int32)
        z16f = jnp.zeros((_L,), jnp.float32)
        kacc, dacc = lax.fori_loop(0, _L, ex_body, (z16i, z16f))
        outk_v[r, pl.ds(0, _L)] = kacc
        outd_v[r, pl.ds(0, _L)] = dacc
        kacc, dacc = lax.fori_loop(_L, MAX_NBR, ex_body, (z16i, z16f))
        outk_v[r, pl.ds(_L, _L)] = kacc
        outd_v[r, pl.ds(_L, _L)] = dacc
        return 0

    lax.fori_loop(0, _RPW, row_body, 0)
    pltpu.sync_copy(outk_v, outk_hbm.at[pl.ds(base, _RPW)])
    pltpu.sync_copy(outd_v, outd_hbm.at[pl.ds(base, _RPW)])


_sc_select = functools.partial(
    pl.kernel,
    out_type=[jax.ShapeDtypeStruct((B * T, MAX_NBR), jnp.int32),
              jax.ShapeDtypeStruct((B * T, MAX_NBR), jnp.float32)],
    mesh=plsc.VectorSubcoreMesh(core_axis_name="c", subcore_axis_name="s"),
    scratch_types=[
        pltpu.VMEM((NCAND,), jnp.float32),
        pltpu.VMEM((NCAND,), jnp.float32),
        pltpu.VMEM((NCAND,), jnp.float32),
        pltpu.VMEM((_RPW, _L), jnp.float32),
        pltpu.VMEM((_RPW, _L), jnp.float32),
        pltpu.VMEM((_RPW, _L), jnp.float32),
        pltpu.VMEM((NCAND + _L,), jnp.float32),
        pltpu.VMEM((NCAND + _L,), jnp.int32),
        pltpu.VMEM((_NCH + _L,), jnp.float32),
        pltpu.VMEM((_RPW, MAX_NBR), jnp.int32),
        pltpu.VMEM((_RPW, MAX_NBR), jnp.float32),
    ],
    compiler_params=pltpu.CompilerParams(needs_layout_passes=False),
)(_sc_select_body)


def _select_body(p2x_ref, p2y_ref, p2z_ref, px_ref, py_ref, pz_ref,
                 idx_ref, d2_ref):
    p2x = p2x_ref[0, 0, :]
    p2y = p2y_ref[0, 0, :]
    p2z = p2z_ref[0, 0, :]
    px = px_ref[0]
    py = py_ref[0]
    pz = pz_ref[0]
    dx = p2x - px
    dy = p2y - py
    dz = p2z - pz
    d2 = dx * dx + dy * dy + dz * dz
    ok = (d2 <= RADIUS * RADIUS) & (d2 > 1e-4)
    d2m = jnp.where(ok, d2, jnp.inf)
    iota = jax.lax.broadcasted_iota(jnp.int32, (_RG, NCAND), 1)
    idx_cols = []
    d2_cols = []
    for _ in range(MAX_NBR):
        rowmin = jnp.min(d2m, axis=1, keepdims=True)
        cand = jnp.where(d2m == rowmin, iota, jnp.int32(2 ** 30))
        amin = jnp.min(cand, axis=1, keepdims=True)
        idx_cols.append(amin)
        d2_cols.append(rowmin)
        d2m = jnp.where(iota == amin, jnp.inf, d2m)
    idx_ref[0] = jnp.concatenate(idx_cols, axis=1)
    d2_ref[0] = jnp.concatenate(d2_cols, axis=1)


def _edge_body(idx_ref, d2_ref, ai_ref, atb_ref, mulm_ref, biasm_ref,
               means_ref, stds_ref, w1_ref, b1_ref, w2_ref, b2_ref, out_ref):
    k = idx_ref[0]
    d2s = d2_ref[0]
    valid = d2s <= RADIUS * RADIUS
    j = (k // C).astype(jnp.float32)

    iota_t = jax.lax.broadcasted_iota(jnp.int32, (_E, T), 1).astype(jnp.float32)
    oj = (j == iota_t).astype(jnp.float32)
    atb = atb_ref[0]
    a_j = jnp.sum(oj * atb, axis=1, keepdims=True)

    a_i = ai_ref[0]
    iota_k = jax.lax.broadcasted_iota(jnp.int32, (_E, NUM_TYPES), 1).astype(jnp.float32)
    ohi = (a_i == iota_k).astype(jnp.float32)
    rows_mul = jnp.dot(ohi, mulm_ref[...], preferred_element_type=jnp.float32, precision=jax.lax.Precision.HIGHEST)
    rows_bias = jnp.dot(ohi, biasm_ref[...], preferred_element_type=jnp.float32, precision=jax.lax.Precision.HIGHEST)
    ohj = (a_j == iota_k).astype(jnp.float32)
    mul_e = jnp.sum(ohj * rows_mul, axis=1, keepdims=True)
    bias_e = jnp.sum(ohj * rows_bias, axis=1, keepdims=True)

    d2c = jnp.where(valid, d2s, 0.0)
    dist = jnp.sqrt(d2c + 1e-12)
    x = mul_e * dist + bias_e

    std = jnp.abs(stds_ref[...]) + 1e-5
    pre = (x - means_ref[...]) / std
    g = jnp.exp(-0.5 * pre * pre) / (_SQRT_2PI * std)
    h1 = jax.nn.gelu(jnp.dot(g, w1_ref[...], preferred_element_type=jnp.float32, precision=jax.lax.Precision.HIGHEST)
                     + b1_ref[...])
    h = jnp.dot(h1, w2_ref[...], preferred_element_type=jnp.float32, precision=jax.lax.Precision.HIGHEST) + b2_ref[...]
    h = jnp.where(valid, h, 0.0)

    for i in range(_BI):
        hi = h[i * MAX_NBR:(i + 1) * MAX_NBR, :]
        si = oj[i * MAX_NBR:(i + 1) * MAX_NBR, :]
        out_ref[0, :, i, :] = jax.lax.dot_general(
            hi, si, (((0,), (0,)), ((), ())), preferred_element_type=jnp.float32, precision=jax.lax.Precision.HIGHEST)


def kernel(pos, atom_types, natoms, cell, means, stds, mul_w, bias_w,
           W1, b1, W2, b2):
    del natoms
    cross_a2a3 = jnp.cross(cell[:, 1], cell[:, 2])
    cell_vol = jnp.sum(cell[:, 0] * cross_a2a3, axis=-1, keepdims=True)
    crosses = [cross_a2a3, jnp.cross(cell[:, 2], cell[:, 0]),
               jnp.cross(cell[:, 0], cell[:, 1])]
    pbc = (True, True, False)
    rep_caps = []
    for dim in range(3):
        if pbc[dim]:
            inv = jnp.linalg.norm(crosses[dim] / cell_vol, axis=-1)
            rep_caps.append(jnp.minimum(jnp.max(jnp.ceil(RADIUS * inv)), float(MAX_REP)))
        else:
            rep_caps.append(jnp.asarray(0.0, jnp.float32))
    static_reps = [MAX_REP if pbc[dim] else 0 for dim in range(3)]
    cells_per_dim = [np.arange(-r, r + 1, dtype=np.float32) for r in static_reps]
    mg = np.meshgrid(*cells_per_dim, indexing="ij")
    unit_cell = jnp.asarray(np.stack([m.reshape(-1) for m in mg], axis=1))
    cell_ok = jnp.ones((C,), bool)
    for dim in range(3):
        cell_ok = cell_ok & (jnp.abs(unit_cell[:, dim]) <= rep_caps[dim])
    pbc_off = jnp.einsum("ci,bij->bcj", unit_cell, cell)
    pos2 = pos[:, :, None, :] + pbc_off[:, None, :, :]
    pos2 = jnp.where(cell_ok[None, None, :, None], pos2, 1e9)
    pos2 = pos2.reshape(B, 1, NCAND, 3)

    p2x = pos2[..., 0].reshape(B, NCAND)
    p2y = pos2[..., 1].reshape(B, NCAND)
    p2z = pos2[..., 2].reshape(B, NCAND)
    pos_flat = pos.reshape(B * T, 3)
    pxs = jnp.broadcast_to(pos_flat[:, 0:1], (B * T, _L))
    pys = jnp.broadcast_to(pos_flat[:, 1:2], (B * T, _L))
    pzs = jnp.broadcast_to(pos_flat[:, 2:3], (B * T, _L))

    sel_idx, sel_d2 = _sc_select(p2x, p2y, p2z, pxs, pys, pzs)
    nsteps = B * T // _BI
    steps_per_b = T // _BI
    sel_idx = sel_idx.reshape(nsteps, _E, 1)
    sel_d2 = sel_d2.reshape(nsteps, _E, 1)

    atype_f = atom_types.astype(jnp.float32)
    ai = jnp.broadcast_to(atype_f.reshape(B * T, 1), (B * T, MAX_NBR))
    ai = ai.reshape(nsteps, _E, 1)
    atb = atype_f.reshape(B, 1, T)
    mulm = mul_w.reshape(NUM_TYPES, NUM_TYPES)
    biasm = bias_w.reshape(NUM_TYPES, NUM_TYPES)

    out = pl.pallas_call(
        _edge_body,
        grid=(nsteps,),
        in_specs=[
            pl.BlockSpec((1, _E, 1), lambda g: (g, 0, 0)),
            pl.BlockSpec((1, _E, 1), lambda g: (g, 0, 0)),
            pl.BlockSpec((1, _E, 1), lambda g: (g, 0, 0)),
            pl.BlockSpec((1, 1, T), lambda g: (g // steps_per_b, 0, 0)),
            pl.BlockSpec((NUM_TYPES, NUM_TYPES), lambda g: (0, 0)),
            pl.BlockSpec((NUM_TYPES, NUM_TYPES), lambda g: (0, 0)),
            pl.BlockSpec((1, K), lambda g: (0, 0)),
            pl.BlockSpec((1, K), lambda g: (0, 0)),
            pl.BlockSpec((K, K), lambda g: (0, 0)),
            pl.BlockSpec((1, K), lambda g: (0, 0)),
            pl.BlockSpec((K, H), lambda g: (0, 0)),
            pl.BlockSpec((1, H), lambda g: (0, 0)),
        ],
        out_specs=pl.BlockSpec(
            (1, H, _BI, T), lambda g: (g // steps_per_b, 0, g % steps_per_b, 0)),
        out_shape=jax.ShapeDtypeStruct((B, H, T, T), jnp.float32),
    )(sel_idx, sel_d2, ai, atb, mulm, biasm,
      means.reshape(1, K), stds.reshape(1, K), W1, b1.reshape(1, K),
      W2, b2.reshape(1, H))
    return out


# speedup vs baseline: 10.8147x; 1.2429x over previous
import functools
import math

import jax
import jax.numpy as jnp
import numpy as np
from jax import lax
from jax.experimental import pallas as pl
from jax.experimental.pallas import tpu as pltpu
from jax.experimental.pallas import tpu_sc as plsc

B, T, NUM_TYPES, K, H = 4, 256, 128, 128, 32
RADIUS, MAX_NBR, MAX_REP = 5.0, 32, 2
C = (2 * MAX_REP + 1) * (2 * MAX_REP + 1)
NCAND = T * C
_SQRT_2PI = math.sqrt(2.0 * math.pi)

_RG = 8
_BI = 32
_E = _BI * MAX_NBR


_L = 16
_NW = 32
_RPW = (B * T) // _NW
_NCH = NCAND // _L
_R2 = RADIUS * RADIUS


def _sc_select_body(p2x_hbm, p2y_hbm, p2z_hbm, pxs_hbm, pys_hbm, pzs_hbm,
                    nav_hbm, outk_hbm, outd_hbm,
                    p2x, p2y, p2z, pxs, pys, pzs, nav_v,
                    cd2, ck, cmin, outk_v, outd_v):
    cid = lax.axis_index("c")
    sid = lax.axis_index("s")
    wid = sid * 2 + cid
    base = wid * _RPW
    b = base // T

    pltpu.sync_copy(p2x_hbm.at[b], p2x)
    pltpu.sync_copy(p2y_hbm.at[b], p2y)
    pltpu.sync_copy(p2z_hbm.at[b], p2z)
    pltpu.sync_copy(pxs_hbm.at[pl.ds(base, _RPW)], pxs)
    pltpu.sync_copy(pys_hbm.at[pl.ds(base, _RPW)], pys)
    pltpu.sync_copy(pzs_hbm.at[pl.ds(base, _RPW)], pzs)
    pltpu.sync_copy(nav_hbm, nav_v)

    iota = lax.broadcasted_iota(jnp.int32, (_L,), 0)
    inf16 = jnp.full((_L,), jnp.inf, jnp.float32)
    nch_scan = nav_v[pl.ds(0, _L)][0] * (T // _L)

    def row_body(r, _):
        px = pxs[r]
        py = pys[r]
        pz = pzs[r]

        def ch_body(ci, cnt):
            off = ci * _L
            dx = p2x[pl.ds(off, _L)] - px
            dy = p2y[pl.ds(off, _L)] - py
            dz = p2z[pl.ds(off, _L)] - pz
            d2 = dx * dx + dy * dy + dz * dz
            m = (d2 <= _R2) & (d2 > 1e-4)
            plsc.store_compressed(cd2.at[pl.ds(cnt, _L)], d2, mask=m)
            plsc.store_compressed(ck.at[pl.ds(cnt, _L)], (off + iota) & (T - 1),
                                  mask=m)
            return cnt + jnp.sum(m.astype(jnp.int32))

        cnt = lax.fori_loop(0, nch_scan, ch_body, jnp.int32(0))
        cd2[pl.ds(cnt, _L)] = inf16

        nch_c = (cnt + _L - 1) // _L
        lane0 = iota == 0

        def cm_body(ci, _):
            mv = jnp.min(cd2[pl.ds(ci * _L, _L)])
            plsc.store_compressed(cmin.at[pl.ds(ci, _L)],
                                  jnp.full((_L,), mv, jnp.float32), mask=lane0)
            return 0

        lax.fori_loop(0, nch_c, cm_body, 0)
        cmin[pl.ds(nch_c, _L)] = inf16
        ncm = (nch_c + _L - 1) // _L

        def ex_body(n, carry):
            kacc, dacc = carry

            def gm_body(ci, gcarry):
                bv, bi = gcarry
                v = cmin[pl.ds(ci * _L, _L)]
                mv = jnp.min(v)
                lane = jnp.min(plsc.all_reduce_ffs(v == mv))
                better = mv < bv
                return (jnp.where(better, mv, bv),
                        jnp.where(better, ci * _L + lane, bi))

            gmin, fch = lax.fori_loop(0, ncm, gm_body,
                                      (jnp.float32(jnp.inf), jnp.int32(0)))
            coff = fch * _L
            v = cd2[pl.ds(coff, _L)]
            lane = jnp.min(plsc.all_reduce_ffs(v == gmin))
            vk = ck[pl.ds(coff, _L)]
            kval = jnp.min(jnp.where(iota == lane, vk, jnp.int32(2 ** 30)))
            kacc = jnp.where(iota == (n % _L), jnp.full((_L,), kval, jnp.int32), kacc)
            dacc = jnp.where(iota == (n % _L), jnp.full((_L,), gmin, jnp.float32), dacc)
            v2 = jnp.where(iota == lane, jnp.inf, v)
            cd2[pl.ds(coff, _L)] = v2
            plsc.store_compressed(cmin.at[pl.ds(fch, _L)],
                                  jnp.full((_L,), jnp.min(v2), jnp.float32),
                                  mask=lane0)
            return kacc, dacc

        z16i = jnp.zeros((_L,), jnp.int32)
        z16f = jnp.zeros((_L,), jnp.float32)
        kacc, dacc = lax.fori_loop(0, _L, ex_body, (z16i, z16f))
        outk_v[r, pl.ds(0, _L)] = kacc
        outd_v[r, pl.ds(0, _L)] = dacc
        kacc, dacc = lax.fori_loop(_L, MAX_NBR, ex_body, (z16i, z16f))
        outk_v[r, pl.ds(_L, _L)] = kacc
        outd_v[r, pl.ds(_L, _L)] = dacc
        return 0

    lax.fori_loop(0, _RPW, row_body, 0)
    pltpu.sync_copy(outk_v, outk_hbm.at[pl.ds(base, _RPW)])
    pltpu.sync_copy(outd_v, outd_hbm.at[pl.ds(base, _RPW)])


_sc_select = functools.partial(
    pl.kernel,
    out_type=[jax.ShapeDtypeStruct((B * T, MAX_NBR), jnp.int32),
              jax.ShapeDtypeStruct((B * T, MAX_NBR), jnp.float32)],
    mesh=plsc.VectorSubcoreMesh(core_axis_name="c", subcore_axis_name="s"),
    scratch_types=[
        pltpu.VMEM((NCAND,), jnp.float32),
        pltpu.VMEM((NCAND,), jnp.float32),
        pltpu.VMEM((NCAND,), jnp.float32),
        pltpu.VMEM((_RPW, _L), jnp.float32),
        pltpu.VMEM((_RPW, _L), jnp.float32),
        pltpu.VMEM((_RPW, _L), jnp.float32),
        pltpu.VMEM((_L,), jnp.int32),
        pltpu.VMEM((NCAND + _L,), jnp.float32),
        pltpu.VMEM((NCAND + _L,), jnp.int32),
        pltpu.VMEM((_NCH + _L,), jnp.float32),
        pltpu.VMEM((_RPW, MAX_NBR), jnp.int32),
        pltpu.VMEM((_RPW, MAX_NBR), jnp.float32),
    ],
    compiler_params=pltpu.CompilerParams(needs_layout_passes=False),
)(_sc_select_body)


def _select_body(p2x_ref, p2y_ref, p2z_ref, px_ref, py_ref, pz_ref,
                 idx_ref, d2_ref):
    p2x = p2x_ref[0, 0, :]
    p2y = p2y_ref[0, 0, :]
    p2z = p2z_ref[0, 0, :]
    px = px_ref[0]
    py = py_ref[0]
    pz = pz_ref[0]
    dx = p2x - px
    dy = p2y - py
    dz = p2z - pz
    d2 = dx * dx + dy * dy + dz * dz
    ok = (d2 <= RADIUS * RADIUS) & (d2 > 1e-4)
    d2m = jnp.where(ok, d2, jnp.inf)
    iota = jax.lax.broadcasted_iota(jnp.int32, (_RG, NCAND), 1)
    idx_cols = []
    d2_cols = []
    for _ in range(MAX_NBR):
        rowmin = jnp.min(d2m, axis=1, keepdims=True)
        cand = jnp.where(d2m == rowmin, iota, jnp.int32(2 ** 30))
        amin = jnp.min(cand, axis=1, keepdims=True)
        idx_cols.append(amin)
        d2_cols.append(rowmin)
        d2m = jnp.where(iota == amin, jnp.inf, d2m)
    idx_ref[0] = jnp.concatenate(idx_cols, axis=1)
    d2_ref[0] = jnp.concatenate(d2_cols, axis=1)


def _edge_body(idx_ref, d2_ref, ai_ref, atb_ref, mulm_ref, biasm_ref,
               means_ref, stds_ref, w1_ref, b1_ref, w2_ref, b2_ref, out_ref):
    k = idx_ref[0]
    d2s = d2_ref[0]
    valid = d2s <= RADIUS * RADIUS
    j = k.astype(jnp.float32)

    iota_t = jax.lax.broadcasted_iota(jnp.int32, (_E, T), 1).astype(jnp.float32)
    oj = (j == iota_t).astype(jnp.float32)
    atb = atb_ref[0]
    a_j = jnp.sum(oj * atb, axis=1, keepdims=True)

    a_i = ai_ref[0]
    iota_k = jax.lax.broadcasted_iota(jnp.int32, (_E, NUM_TYPES), 1).astype(jnp.float32)
    ohi = (a_i == iota_k).astype(jnp.float32)
    rows_mul = jnp.dot(ohi, mulm_ref[...], preferred_element_type=jnp.float32, precision=jax.lax.Precision.HIGHEST)
    rows_bias = jnp.dot(ohi, biasm_ref[...], preferred_element_type=jnp.float32, precision=jax.lax.Precision.HIGHEST)
    ohj = (a_j == iota_k).astype(jnp.float32)
    mul_e = jnp.sum(ohj * rows_mul, axis=1, keepdims=True)
    bias_e = jnp.sum(ohj * rows_bias, axis=1, keepdims=True)

    d2c = jnp.where(valid, d2s, 0.0)
    dist = jnp.sqrt(d2c + 1e-12)
    x = mul_e * dist + bias_e

    std = jnp.abs(stds_ref[...]) + 1e-5
    pre = (x - means_ref[...]) / std
    g = jnp.exp(-0.5 * pre * pre) / (_SQRT_2PI * std)
    h1 = jax.nn.gelu(jnp.dot(g, w1_ref[...], preferred_element_type=jnp.float32, precision=jax.lax.Precision.HIGHEST)
                     + b1_ref[...])
    h = jnp.dot(h1, w2_ref[...], preferred_element_type=jnp.float32, precision=jax.lax.Precision.HIGHEST) + b2_ref[...]
    h = jnp.where(valid, h, 0.0)

    for i in range(_BI):
        hi = h[i * MAX_NBR:(i + 1) * MAX_NBR, :]
        si = oj[i * MAX_NBR:(i + 1) * MAX_NBR, :]
        out_ref[0, :, i, :] = jax.lax.dot_general(
            hi, si, (((0,), (0,)), ((), ())), preferred_element_type=jnp.float32, precision=jax.lax.Precision.HIGHEST)


def kernel(pos, atom_types, natoms, cell, means, stds, mul_w, bias_w,
           W1, b1, W2, b2):
    del natoms
    cross_a2a3 = jnp.cross(cell[:, 1], cell[:, 2])
    cell_vol = jnp.sum(cell[:, 0] * cross_a2a3, axis=-1, keepdims=True)
    crosses = [cross_a2a3, jnp.cross(cell[:, 2], cell[:, 0]),
               jnp.cross(cell[:, 0], cell[:, 1])]
    pbc = (True, True, False)
    rep_caps = []
    for dim in range(3):
        if pbc[dim]:
            inv = jnp.linalg.norm(crosses[dim] / cell_vol, axis=-1)
            rep_caps.append(jnp.minimum(jnp.max(jnp.ceil(RADIUS * inv)), float(MAX_REP)))
        else:
            rep_caps.append(jnp.asarray(0.0, jnp.float32))
    static_reps = [MAX_REP if pbc[dim] else 0 for dim in range(3)]
    cells_per_dim = [np.arange(-r, r + 1, dtype=np.float32) for r in static_reps]
    mg = np.meshgrid(*cells_per_dim, indexing="ij")
    unit_cell = jnp.asarray(np.stack([m.reshape(-1) for m in mg], axis=1))
    cell_ok = jnp.ones((C,), bool)
    for dim in range(3):
        cell_ok = cell_ok & (jnp.abs(unit_cell[:, dim]) <= rep_caps[dim])
    pbc_off = jnp.einsum("ci,bij->bcj", unit_cell, cell)
    order = jnp.argsort(jnp.logical_not(cell_ok).astype(jnp.int32), stable=True)
    n_alive = jnp.sum(cell_ok.astype(jnp.int32))
    pbc_off = pbc_off[:, order, :]
    pos2 = pbc_off[:, :, None, :] + pos[:, None, :, :]
    pos2 = pos2.reshape(B, NCAND, 3)

    p2x = pos2[..., 0]
    p2y = pos2[..., 1]
    p2z = pos2[..., 2]
    pos_flat = pos.reshape(B * T, 3)
    pxs = jnp.broadcast_to(pos_flat[:, 0:1], (B * T, _L))
    pys = jnp.broadcast_to(pos_flat[:, 1:2], (B * T, _L))
    pzs = jnp.broadcast_to(pos_flat[:, 2:3], (B * T, _L))
    nav = jnp.full((_L,), n_alive, jnp.int32)

    sel_idx, sel_d2 = _sc_select(p2x, p2y, p2z, pxs, pys, pzs, nav)
    nsteps = B * T // _BI
    steps_per_b = T // _BI
    sel_idx = sel_idx.reshape(nsteps, _E, 1)
    sel_d2 = sel_d2.reshape(nsteps, _E, 1)

    atype_f = atom_types.astype(jnp.float32)
    ai = jnp.broadcast_to(atype_f.reshape(B * T, 1), (B * T, MAX_NBR))
    ai = ai.reshape(nsteps, _E, 1)
    atb = atype_f.reshape(B, 1, T)
    mulm = mul_w.reshape(NUM_TYPES, NUM_TYPES)
    biasm = bias_w.reshape(NUM_TYPES, NUM_TYPES)

    out = pl.pallas_call(
        _edge_body,
        grid=(nsteps,),
        in_specs=[
            pl.BlockSpec((1, _E, 1), lambda g: (g, 0, 0)),
            pl.BlockSpec((1, _E, 1), lambda g: (g, 0, 0)),
            pl.BlockSpec((1, _E, 1), lambda g: (g, 0, 0)),
            pl.BlockSpec((1, 1, T), lambda g: (g // steps_per_b, 0, 0)),
            pl.BlockSpec((NUM_TYPES, NUM_TYPES), lambda g: (0, 0)),
            pl.BlockSpec((NUM_TYPES, NUM_TYPES), lambda g: (0, 0)),
            pl.BlockSpec((1, K), lambda g: (0, 0)),
            pl.BlockSpec((1, K), lambda g: (0, 0)),
            pl.BlockSpec((K, K), lambda g: (0, 0)),
            pl.BlockSpec((1, K), lambda g: (0, 0)),
            pl.BlockSpec((K, H), lambda g: (0, 0)),
            pl.BlockSpec((1, H), lambda g: (0, 0)),
        ],
        out_specs=pl.BlockSpec(
            (1, H, _BI, T), lambda g: (g // steps_per_b, 0, g % steps_per_b, 0)),
        out_shape=jax.ShapeDtypeStruct((B, H, T, T), jnp.float32),
    )(sel_idx, sel_d2, ai, atb, mulm, biasm,
      means.reshape(1, K), stds.reshape(1, K), W1, b1.reshape(1, K),
      W2, b2.reshape(1, H))
    return out


# speedup vs baseline: 11.1157x; 1.0278x over previous
import functools
import math

import jax
import jax.numpy as jnp
import numpy as np
from jax import lax
from jax.experimental import pallas as pl
from jax.experimental.pallas import tpu as pltpu
from jax.experimental.pallas import tpu_sc as plsc

B, T, NUM_TYPES, K, H = 4, 256, 128, 128, 32
RADIUS, MAX_NBR, MAX_REP = 5.0, 32, 2
C = (2 * MAX_REP + 1) * (2 * MAX_REP + 1)
NCAND = T * C
_SQRT_2PI = math.sqrt(2.0 * math.pi)

_RG = 8
_BI = 32
_E = _BI * MAX_NBR


_L = 16
_NW = 32
_RPW = (B * T) // _NW
_NCH = NCAND // _L
_R2 = RADIUS * RADIUS


def _sc_select_body(p2x_hbm, p2y_hbm, p2z_hbm, pxs_hbm, pys_hbm, pzs_hbm,
                    nav_hbm, outk_hbm, outd_hbm,
                    p2x, p2y, p2z, pxs, pys, pzs, nav_v,
                    cd2, ck, cmin, outk_v, outd_v):
    cid = lax.axis_index("c")
    sid = lax.axis_index("s")
    wid = sid * 2 + cid
    base = wid * _RPW
    b = base // T

    pltpu.sync_copy(p2x_hbm.at[b], p2x)
    pltpu.sync_copy(p2y_hbm.at[b], p2y)
    pltpu.sync_copy(p2z_hbm.at[b], p2z)
    pltpu.sync_copy(pxs_hbm.at[pl.ds(base, _RPW)], pxs)
    pltpu.sync_copy(pys_hbm.at[pl.ds(base, _RPW)], pys)
    pltpu.sync_copy(pzs_hbm.at[pl.ds(base, _RPW)], pzs)
    pltpu.sync_copy(nav_hbm, nav_v)

    iota = lax.broadcasted_iota(jnp.int32, (_L,), 0)
    inf16 = jnp.full((_L,), jnp.inf, jnp.float32)
    nch_scan = nav_v[pl.ds(0, _L)][0] * (T // _L)

    def row_body(r, _):
        px = pxs[r]
        py = pys[r]
        pz = pzs[r]

        def ch_body(ci, cnt):
            off = ci * _L
            dx = p2x[pl.ds(off, _L)] - px
            dy = p2y[pl.ds(off, _L)] - py
            dz = p2z[pl.ds(off, _L)] - pz
            d2 = dx * dx + dy * dy + dz * dz
            m = (d2 <= _R2) & (d2 > 1e-4)
            plsc.store_compressed(cd2.at[pl.ds(cnt, _L)], d2, mask=m)
            plsc.store_compressed(ck.at[pl.ds(cnt, _L)], (off + iota) & (T - 1),
                                  mask=m)
            return cnt + jnp.sum(m.astype(jnp.int32))

        cnt = lax.fori_loop(0, nch_scan, ch_body, jnp.int32(0))
        cd2[pl.ds(cnt, _L)] = inf16

        nch_c = (cnt + _L - 1) // _L
        lane0 = iota == 0

        def cm_body(ci, _):
            mv = jnp.min(cd2[pl.ds(ci * _L, _L)])
            plsc.store_compressed(cmin.at[pl.ds(ci, _L)],
                                  jnp.full((_L,), mv, jnp.float32), mask=lane0)
            return 0

        lax.fori_loop(0, nch_c, cm_body, 0)
        cmin[pl.ds(nch_c, _L)] = inf16
        ncm = (nch_c + _L - 1) // _L

        def ex_body(n, carry):
            kacc, dacc = carry

            def gm_body(ci, gcarry):
                bv, bi = gcarry
                v = cmin[pl.ds(ci * _L, _L)]
                mv = jnp.min(v)
                lane = jnp.min(plsc.all_reduce_ffs(v == mv))
                better = mv < bv
                return (jnp.where(better, mv, bv),
                        jnp.where(better, ci * _L + lane, bi))

            gmin, fch = lax.fori_loop(0, ncm, gm_body,
                                      (jnp.float32(jnp.inf), jnp.int32(0)))
            coff = fch * _L
            v = cd2[pl.ds(coff, _L)]
            lane = jnp.min(plsc.all_reduce_ffs(v == gmin))
            vk = ck[pl.ds(coff, _L)]
            kval = jnp.min(jnp.where(iota == lane, vk, jnp.int32(2 ** 30)))
            kacc = jnp.where(iota == (n % _L), jnp.full((_L,), kval, jnp.int32), kacc)
            dacc = jnp.where(iota == (n % _L), jnp.full((_L,), gmin, jnp.float32), dacc)
            v2 = jnp.where(iota == lane, jnp.inf, v)
            cd2[pl.ds(coff, _L)] = v2
            plsc.store_compressed(cmin.at[pl.ds(fch, _L)],
                                  jnp.full((_L,), jnp.min(v2), jnp.float32),
                                  mask=lane0)
            return kacc, dacc

        z16i = jnp.zeros((_L,), jnp.int32)
        z16f = jnp.zeros((_L,), jnp.float32)
        kacc, dacc = lax.fori_loop(0, _L, ex_body, (z16i, z16f))
        outk_v[r, pl.ds(0, _L)] = kacc
        outd_v[r, pl.ds(0, _L)] = dacc
        kacc, dacc = lax.fori_loop(_L, MAX_NBR, ex_body, (z16i, z16f))
        outk_v[r, pl.ds(_L, _L)] = kacc
        outd_v[r, pl.ds(_L, _L)] = dacc
        return 0

    lax.fori_loop(0, _RPW, row_body, 0)
    pltpu.sync_copy(outk_v, outk_hbm.at[pl.ds(base, _RPW)])
    pltpu.sync_copy(outd_v, outd_hbm.at[pl.ds(base, _RPW)])


_sc_select = functools.partial(
    pl.kernel,
    out_type=[jax.ShapeDtypeStruct((B * T, MAX_NBR), jnp.int32),
              jax.ShapeDtypeStruct((B * T, MAX_NBR), jnp.float32)],
    mesh=plsc.VectorSubcoreMesh(core_axis_name="c", subcore_axis_name="s"),
    scratch_types=[
        pltpu.VMEM((NCAND,), jnp.float32),
        pltpu.VMEM((NCAND,), jnp.float32),
        pltpu.VMEM((NCAND,), jnp.float32),
        pltpu.VMEM((_RPW, _L), jnp.float32),
        pltpu.VMEM((_RPW, _L), jnp.float32),
        pltpu.VMEM((_RPW, _L), jnp.float32),
        pltpu.VMEM((_L,), jnp.int32),
        pltpu.VMEM((NCAND + _L,), jnp.float32),
        pltpu.VMEM((NCAND + _L,), jnp.int32),
        pltpu.VMEM((_NCH + _L,), jnp.float32),
        pltpu.VMEM((_RPW, MAX_NBR), jnp.int32),
        pltpu.VMEM((_RPW, MAX_NBR), jnp.float32),
    ],
    compiler_params=pltpu.CompilerParams(needs_layout_passes=False),
)(_sc_select_body)


def _select_body(p2x_ref, p2y_ref, p2z_ref, px_ref, py_ref, pz_ref,
                 idx_ref, d2_ref):
    p2x = p2x_ref[0, 0, :]
    p2y = p2y_ref[0, 0, :]
    p2z = p2z_ref[0, 0, :]
    px = px_ref[0]
    py = py_ref[0]
    pz = pz_ref[0]
    dx = p2x - px
    dy = p2y - py
    dz = p2z - pz
    d2 = dx * dx + dy * dy + dz * dz
    ok = (d2 <= RADIUS * RADIUS) & (d2 > 1e-4)
    d2m = jnp.where(ok, d2, jnp.inf)
    iota = jax.lax.broadcasted_iota(jnp.int32, (_RG, NCAND), 1)
    idx_cols = []
    d2_cols = []
    for _ in range(MAX_NBR):
        rowmin = jnp.min(d2m, axis=1, keepdims=True)
        cand = jnp.where(d2m == rowmin, iota, jnp.int32(2 ** 30))
        amin = jnp.min(cand, axis=1, keepdims=True)
        idx_cols.append(amin)
        d2_cols.append(rowmin)
        d2m = jnp.where(iota == amin, jnp.inf, d2m)
    idx_ref[0] = jnp.concatenate(idx_cols, axis=1)
    d2_ref[0] = jnp.concatenate(d2_cols, axis=1)


def _edge_body(idx_ref, d2_ref, ai_ref, atb_ref, mulm_ref, biasm_ref,
               means_ref, stds_ref, w1_ref, b1_ref, w2_ref, b2_ref, out_ref):
    k = idx_ref[0]
    d2s = d2_ref[0]
    valid = d2s <= RADIUS * RADIUS
    j = k.astype(jnp.float32)

    iota_t = jax.lax.broadcasted_iota(jnp.int32, (_E, T), 1).astype(jnp.float32)
    oj = (j == iota_t).astype(jnp.float32)
    atb = atb_ref[0]
    a_j = jnp.sum(oj * atb, axis=1, keepdims=True)

    a_i = ai_ref[0]
    iota_k = jax.lax.broadcasted_iota(jnp.int32, (_E, NUM_TYPES), 1).astype(jnp.float32)
    ohi = (a_i == iota_k).astype(jnp.float32)
    rows_mul = jnp.dot(ohi, mulm_ref[...], preferred_element_type=jnp.float32, precision=jax.lax.Precision.HIGHEST)
    rows_bias = jnp.dot(ohi, biasm_ref[...], preferred_element_type=jnp.float32, precision=jax.lax.Precision.HIGHEST)
    ohj = (a_j == iota_k).astype(jnp.float32)
    mul_e = jnp.sum(ohj * rows_mul, axis=1, keepdims=True)
    bias_e = jnp.sum(ohj * rows_bias, axis=1, keepdims=True)

    d2c = jnp.where(valid, d2s, 0.0)
    dist = jnp.sqrt(d2c + 1e-12)
    x = mul_e * dist + bias_e

    std = jnp.abs(stds_ref[...]) + 1e-5
    pre = (x - means_ref[...]) / std
    g = jnp.exp(-0.5 * pre * pre) / (_SQRT_2PI * std)
    h1 = jax.nn.gelu(jnp.dot(g, w1_ref[...], preferred_element_type=jnp.float32, precision=jax.lax.Precision.HIGHEST)
                     + b1_ref[...])
    h = jnp.dot(h1, w2_ref[...], preferred_element_type=jnp.float32, precision=jax.lax.Precision.HIGHEST) + b2_ref[...]
    h = jnp.where(valid, h, 0.0)
    ht = h.T

    for i in range(_BI):
        hi = ht[:, i * MAX_NBR:(i + 1) * MAX_NBR]
        si = oj[i * MAX_NBR:(i + 1) * MAX_NBR, :]
        out_ref[0, :, i, :] = jnp.dot(
            hi, si, preferred_element_type=jnp.float32,
            precision=jax.lax.Precision.HIGHEST)


def kernel(pos, atom_types, natoms, cell, means, stds, mul_w, bias_w,
           W1, b1, W2, b2):
    del natoms
    cross_a2a3 = jnp.cross(cell[:, 1], cell[:, 2])
    cell_vol = jnp.sum(cell[:, 0] * cross_a2a3, axis=-1, keepdims=True)
    crosses = [cross_a2a3, jnp.cross(cell[:, 2], cell[:, 0]),
               jnp.cross(cell[:, 0], cell[:, 1])]
    pbc = (True, True, False)
    rep_caps = []
    for dim in range(3):
        if pbc[dim]:
            inv = jnp.linalg.norm(crosses[dim] / cell_vol, axis=-1)
            rep_caps.append(jnp.minimum(jnp.max(jnp.ceil(RADIUS * inv)), float(MAX_REP)))
        else:
            rep_caps.append(jnp.asarray(0.0, jnp.float32))
    static_reps = [MAX_REP if pbc[dim] else 0 for dim in range(3)]
    cells_per_dim = [np.arange(-r, r + 1, dtype=np.float32) for r in static_reps]
    mg = np.meshgrid(*cells_per_dim, indexing="ij")
    unit_cell = jnp.asarray(np.stack([m.reshape(-1) for m in mg], axis=1))
    cell_ok = jnp.ones((C,), bool)
    for dim in range(3):
        cell_ok = cell_ok & (jnp.abs(unit_cell[:, dim]) <= rep_caps[dim])
    pbc_off = jnp.einsum("ci,bij->bcj", unit_cell, cell)
    order = jnp.argsort(jnp.logical_not(cell_ok).astype(jnp.int32), stable=True)
    n_alive = jnp.sum(cell_ok.astype(jnp.int32))
    pbc_off = pbc_off[:, order, :]
    pos2 = pbc_off[:, :, None, :] + pos[:, None, :, :]
    pos2 = pos2.reshape(B, NCAND, 3)

    p2x = pos2[..., 0]
    p2y = pos2[..., 1]
    p2z = pos2[..., 2]
    pos_flat = pos.reshape(B * T, 3)
    pxs = jnp.broadcast_to(pos_flat[:, 0:1], (B * T, _L))
    pys = jnp.broadcast_to(pos_flat[:, 1:2], (B * T, _L))
    pzs = jnp.broadcast_to(pos_flat[:, 2:3], (B * T, _L))
    nav = jnp.full((_L,), n_alive, jnp.int32)

    sel_idx, sel_d2 = _sc_select(p2x, p2y, p2z, pxs, pys, pzs, nav)
    nsteps = B * T // _BI
    steps_per_b = T // _BI
    sel_idx = sel_idx.reshape(nsteps, _E, 1)
    sel_d2 = sel_d2.reshape(nsteps, _E, 1)

    atype_f = atom_types.astype(jnp.float32)
    ai = jnp.broadcast_to(atype_f.reshape(B * T, 1), (B * T, MAX_NBR))
    ai = ai.reshape(nsteps, _E, 1)
    atb = atype_f.reshape(B, 1, T)
    mulm = mul_w.reshape(NUM_TYPES, NUM_TYPES)
    biasm = bias_w.reshape(NUM_TYPES, NUM_TYPES)

    out = pl.pallas_call(
        _edge_body,
        grid=(nsteps,),
        in_specs=[
            pl.BlockSpec((1, _E, 1), lambda g: (g, 0, 0)),
            pl.BlockSpec((1, _E, 1), lambda g: (g, 0, 0)),
            pl.BlockSpec((1, _E, 1), lambda g: (g, 0, 0)),
            pl.BlockSpec((1, 1, T), lambda g: (g // steps_per_b, 0, 0)),
            pl.BlockSpec((NUM_TYPES, NUM_TYPES), lambda g: (0, 0)),
            pl.BlockSpec((NUM_TYPES, NUM_TYPES), lambda g: (0, 0)),
            pl.BlockSpec((1, K), lambda g: (0, 0)),
            pl.BlockSpec((1, K), lambda g: (0, 0)),
            pl.BlockSpec((K, K), lambda g: (0, 0)),
            pl.BlockSpec((1, K), lambda g: (0, 0)),
            pl.BlockSpec((K, H), lambda g: (0, 0)),
            pl.BlockSpec((1, H), lambda g: (0, 0)),
        ],
        out_specs=pl.BlockSpec(
            (1, H, _BI, T), lambda g: (g // steps_per_b, 0, g % steps_per_b, 0)),
        out_shape=jax.ShapeDtypeStruct((B, H, T, T), jnp.float32),
    )(sel_idx, sel_d2, ai, atb, mulm, biasm,
      means.reshape(1, K), stds.reshape(1, K), W1, b1.reshape(1, K),
      W2, b2.reshape(1, H))
    return out


# speedup vs baseline: 11.6523x; 1.0483x over previous
import functools
import math

import jax
import jax.numpy as jnp
import numpy as np
from jax import lax
from jax.experimental import pallas as pl
from jax.experimental.pallas import tpu as pltpu
from jax.experimental.pallas import tpu_sc as plsc

B, T, NUM_TYPES, K, H = 4, 256, 128, 128, 32
RADIUS, MAX_NBR, MAX_REP = 5.0, 32, 2
C = (2 * MAX_REP + 1) * (2 * MAX_REP + 1)
NCAND = T * C
_SQRT_2PI = math.sqrt(2.0 * math.pi)

_RG = 8
_BI = 32
_E = _BI * MAX_NBR


_L = 16
_NW = 32
_RPW = (B * T) // _NW
_NCH = NCAND // _L
_R2 = RADIUS * RADIUS


def _sc_select_body(p2x_hbm, p2y_hbm, p2z_hbm, pxs_hbm, pys_hbm, pzs_hbm,
                    nav_hbm, at_hbm, mul_hbm, bias_hbm,
                    outk_hbm, outd_hbm, outm_hbm, outb_hbm,
                    p2x, p2y, p2z, pxs, pys, pzs, nav_v,
                    atb_vm, mul_vm, bias_vm,
                    cd2, ck, cmin, outk_v, outd_v, outm_v, outb_v):
    cid = lax.axis_index("c")
    sid = lax.axis_index("s")
    wid = sid * 2 + cid
    base = wid * _RPW
    b = base // T

    pltpu.sync_copy(p2x_hbm.at[b], p2x)
    pltpu.sync_copy(p2y_hbm.at[b], p2y)
    pltpu.sync_copy(p2z_hbm.at[b], p2z)
    pltpu.sync_copy(pxs_hbm.at[pl.ds(base, _RPW)], pxs)
    pltpu.sync_copy(pys_hbm.at[pl.ds(base, _RPW)], pys)
    pltpu.sync_copy(pzs_hbm.at[pl.ds(base, _RPW)], pzs)
    pltpu.sync_copy(nav_hbm, nav_v)
    pltpu.sync_copy(at_hbm.at[b], atb_vm)
    pltpu.sync_copy(mul_hbm, mul_vm)
    pltpu.sync_copy(bias_hbm, bias_vm)

    iota = lax.broadcasted_iota(jnp.int32, (_L,), 0)
    inf16 = jnp.full((_L,), jnp.inf, jnp.float32)
    nch_scan = nav_v[pl.ds(0, _L)][0] * (T // _L)

    def row_body(r, _):
        px = pxs[r]
        py = pys[r]
        pz = pzs[r]

        def ch_body(ci, cnt):
            off = ci * _L
            dx = p2x[pl.ds(off, _L)] - px
            dy = p2y[pl.ds(off, _L)] - py
            dz = p2z[pl.ds(off, _L)] - pz
            d2 = dx * dx + dy * dy + dz * dz
            m = (d2 <= _R2) & (d2 > 1e-4)
            plsc.store_compressed(cd2.at[pl.ds(cnt, _L)], d2, mask=m)
            plsc.store_compressed(ck.at[pl.ds(cnt, _L)], (off + iota) & (T - 1),
                                  mask=m)
            return cnt + jnp.sum(m.astype(jnp.int32))

        cnt = lax.fori_loop(0, nch_scan, ch_body, jnp.int32(0))
        cd2[pl.ds(cnt, _L)] = inf16

        nch_c = (cnt + _L - 1) // _L
        lane0 = iota == 0

        def cm_body(ci, _):
            mv = jnp.min(cd2[pl.ds(ci * _L, _L)])
            plsc.store_compressed(cmin.at[pl.ds(ci, _L)],
                                  jnp.full((_L,), mv, jnp.float32), mask=lane0)
            return 0

        lax.fori_loop(0, nch_c, cm_body, 0)
        cmin[pl.ds(nch_c, _L)] = inf16
        ncm = (nch_c + _L - 1) // _L

        def ex_body(n, carry):
            kacc, dacc = carry

            def gm_body(ci, gcarry):
                bv, bi = gcarry
                v = cmin[pl.ds(ci * _L, _L)]
                mv = jnp.min(v)
                lane = jnp.min(plsc.all_reduce_ffs(v == mv))
                better = mv < bv
                return (jnp.where(better, mv, bv),
                        jnp.where(better, ci * _L + lane, bi))

            gmin, fch = lax.fori_loop(0, ncm, gm_body,
                                      (jnp.float32(jnp.inf), jnp.int32(0)))
            coff = fch * _L
            v = cd2[pl.ds(coff, _L)]
            lane = jnp.min(plsc.all_reduce_ffs(v == gmin))
            vk = ck[pl.ds(coff, _L)]
            kval = jnp.min(jnp.where(iota == lane, vk, jnp.int32(2 ** 30)))
            kacc = jnp.where(iota == (n % _L), jnp.full((_L,), kval, jnp.int32), kacc)
            dacc = jnp.where(iota == (n % _L), jnp.full((_L,), gmin, jnp.float32), dacc)
            v2 = jnp.where(iota == lane, jnp.inf, v)
            cd2[pl.ds(coff, _L)] = v2
            plsc.store_compressed(cmin.at[pl.ds(fch, _L)],
                                  jnp.full((_L,), jnp.min(v2), jnp.float32),
                                  mask=lane0)
            return kacc, dacc

        lrow = base - b * T + r
        a_i = plsc.load_gather(atb_vm, [jnp.full((_L,), lrow, jnp.int32)])

        def emit_half(h0, kacc, dacc):
            outk_v[r, pl.ds(h0, _L)] = kacc
            outd_v[r, pl.ds(h0, _L)] = dacc
            a_j = plsc.load_gather(atb_vm, [kacc & (T - 1)])
            etype = a_i * NUM_TYPES + a_j
            outm_v[r, pl.ds(h0, _L)] = plsc.load_gather(mul_vm, [etype])
            outb_v[r, pl.ds(h0, _L)] = plsc.load_gather(bias_vm, [etype])

        z16i = jnp.zeros((_L,), jnp.int32)
        z16f = jnp.zeros((_L,), jnp.float32)
        kacc, dacc = lax.fori_loop(0, _L, ex_body, (z16i, z16f))
        emit_half(0, kacc, dacc)
        kacc, dacc = lax.fori_loop(_L, MAX_NBR, ex_body, (z16i, z16f))
        emit_half(_L, kacc, dacc)
        return 0

    lax.fori_loop(0, _RPW, row_body, 0)
    pltpu.sync_copy(outk_v, outk_hbm.at[pl.ds(base, _RPW)])
    pltpu.sync_copy(outd_v, outd_hbm.at[pl.ds(base, _RPW)])
    pltpu.sync_copy(outm_v, outm_hbm.at[pl.ds(base, _RPW)])
    pltpu.sync_copy(outb_v, outb_hbm.at[pl.ds(base, _RPW)])


_sc_select = functools.partial(
    pl.kernel,
    out_type=[jax.ShapeDtypeStruct((B * T, MAX_NBR), jnp.int32),
              jax.ShapeDtypeStruct((B * T, MAX_NBR), jnp.float32),
              jax.ShapeDtypeStruct((B * T, MAX_NBR), jnp.float32),
              jax.ShapeDtypeStruct((B * T, MAX_NBR), jnp.float32)],
    mesh=plsc.VectorSubcoreMesh(core_axis_name="c", subcore_axis_name="s"),
    scratch_types=[
        pltpu.VMEM((NCAND,), jnp.float32),
        pltpu.VMEM((NCAND,), jnp.float32),
        pltpu.VMEM((NCAND,), jnp.float32),
        pltpu.VMEM((_RPW, _L), jnp.float32),
        pltpu.VMEM((_RPW, _L), jnp.float32),
        pltpu.VMEM((_RPW, _L), jnp.float32),
        pltpu.VMEM((_L,), jnp.int32),
        pltpu.VMEM((T,), jnp.int32),
        pltpu.VMEM((NUM_TYPES * NUM_TYPES,), jnp.float32),
        pltpu.VMEM((NUM_TYPES * NUM_TYPES,), jnp.float32),
        pltpu.VMEM((NCAND + _L,), jnp.float32),
        pltpu.VMEM((NCAND + _L,), jnp.int32),
        pltpu.VMEM((_NCH + _L,), jnp.float32),
        pltpu.VMEM((_RPW, MAX_NBR), jnp.int32),
        pltpu.VMEM((_RPW, MAX_NBR), jnp.float32),
        pltpu.VMEM((_RPW, MAX_NBR), jnp.float32),
        pltpu.VMEM((_RPW, MAX_NBR), jnp.float32),
    ],
    compiler_params=pltpu.CompilerParams(needs_layout_passes=False),
)(_sc_select_body)


def _select_body(p2x_ref, p2y_ref, p2z_ref, px_ref, py_ref, pz_ref,
                 idx_ref, d2_ref):
    p2x = p2x_ref[0, 0, :]
    p2y = p2y_ref[0, 0, :]
    p2z = p2z_ref[0, 0, :]
    px = px_ref[0]
    py = py_ref[0]
    pz = pz_ref[0]
    dx = p2x - px
    dy = p2y - py
    dz = p2z - pz
    d2 = dx * dx + dy * dy + dz * dz
    ok = (d2 <= RADIUS * RADIUS) & (d2 > 1e-4)
    d2m = jnp.where(ok, d2, jnp.inf)
    iota = jax.lax.broadcasted_iota(jnp.int32, (_RG, NCAND), 1)
    idx_cols = []
    d2_cols = []
    for _ in range(MAX_NBR):
        rowmin = jnp.min(d2m, axis=1, keepdims=True)
        cand = jnp.where(d2m == rowmin, iota, jnp.int32(2 ** 30))
        amin = jnp.min(cand, axis=1, keepdims=True)
        idx_cols.append(amin)
        d2_cols.append(rowmin)
        d2m = jnp.where(iota == amin, jnp.inf, d2m)
    idx_ref[0] = jnp.concatenate(idx_cols, axis=1)
    d2_ref[0] = jnp.concatenate(d2_cols, axis=1)


def _edge_body(idx_ref, d2_ref, mule_ref, biase_ref,
               means_ref, stds_ref, w1_ref, b1_ref, w2_ref, b2_ref, out_ref):
    k = idx_ref[0]
    d2s = d2_ref[0]
    valid = d2s <= RADIUS * RADIUS
    j = k.astype(jnp.float32)

    iota_t = jax.lax.broadcasted_iota(jnp.int32, (_E, T), 1).astype(jnp.float32)
    oj = (j == iota_t).astype(jnp.float32)
    mul_e = mule_ref[0]
    bias_e = biase_ref[0]

    d2c = jnp.where(valid, d2s, 0.0)
    dist = jnp.sqrt(d2c + 1e-12)
    x = mul_e * dist + bias_e

    std = jnp.abs(stds_ref[...]) + 1e-5
    pre = (x - means_ref[...]) / std
    g = jnp.exp(-0.5 * pre * pre) / (_SQRT_2PI * std)
    h1 = jax.nn.gelu(jnp.dot(g, w1_ref[...], preferred_element_type=jnp.float32, precision=jax.lax.Precision.HIGHEST)
                     + b1_ref[...])
    h = jnp.dot(h1, w2_ref[...], preferred_element_type=jnp.float32, precision=jax.lax.Precision.HIGHEST) + b2_ref[...]
    h = jnp.where(valid, h, 0.0)
    hb = h.astype(jnp.bfloat16).astype(jnp.float32)
    hr = h - hb
    htb = hb.T
    htr = hr.T

    for i in range(_BI):
        sl = slice(i * MAX_NBR, (i + 1) * MAX_NBR)
        si = oj[sl, :]
        out_ref[0, :, i, :] = (
            jnp.dot(htb[:, sl], si, preferred_element_type=jnp.float32)
            + jnp.dot(htr[:, sl], si, preferred_element_type=jnp.float32))


def kernel(pos, atom_types, natoms, cell, means, stds, mul_w, bias_w,
           W1, b1, W2, b2):
    del natoms
    cross_a2a3 = jnp.cross(cell[:, 1], cell[:, 2])
    cell_vol = jnp.sum(cell[:, 0] * cross_a2a3, axis=-1, keepdims=True)
    crosses = [cross_a2a3, jnp.cross(cell[:, 2], cell[:, 0]),
               jnp.cross(cell[:, 0], cell[:, 1])]
    pbc = (True, True, False)
    rep_caps = []
    for dim in range(3):
        if pbc[dim]:
            inv = jnp.linalg.norm(crosses[dim] / cell_vol, axis=-1)
            rep_caps.append(jnp.minimum(jnp.max(jnp.ceil(RADIUS * inv)), float(MAX_REP)))
        else:
            rep_caps.append(jnp.asarray(0.0, jnp.float32))
    static_reps = [MAX_REP if pbc[dim] else 0 for dim in range(3)]
    cells_per_dim = [np.arange(-r, r + 1, dtype=np.float32) for r in static_reps]
    mg = np.meshgrid(*cells_per_dim, indexing="ij")
    unit_cell = jnp.asarray(np.stack([m.reshape(-1) for m in mg], axis=1))
    cell_ok = jnp.ones((C,), bool)
    for dim in range(3):
        cell_ok = cell_ok & (jnp.abs(unit_cell[:, dim]) <= rep_caps[dim])
    pbc_off = jnp.einsum("ci,bij->bcj", unit_cell, cell)
    order = jnp.argsort(jnp.logical_not(cell_ok).astype(jnp.int32), stable=True)
    n_alive = jnp.sum(cell_ok.astype(jnp.int32))
    pbc_off = pbc_off[:, order, :]
    pos2 = pbc_off[:, :, None, :] + pos[:, None, :, :]
    pos2 = pos2.reshape(B, NCAND, 3)

    p2x = pos2[..., 0]
    p2y = pos2[..., 1]
    p2z = pos2[..., 2]
    pos_flat = pos.reshape(B * T, 3)
    pxs = jnp.broadcast_to(pos_flat[:, 0:1], (B * T, _L))
    pys = jnp.broadcast_to(pos_flat[:, 1:2], (B * T, _L))
    pzs = jnp.broadcast_to(pos_flat[:, 2:3], (B * T, _L))
    nav = jnp.full((_L,), n_alive, jnp.int32)

    sel_idx, sel_d2, mul_e, bias_e = _sc_select(
        p2x, p2y, p2z, pxs, pys, pzs, nav, atom_types.astype(jnp.int32),
        mul_w, bias_w)
    nsteps = B * T // _BI
    steps_per_b = T // _BI
    sel_idx = sel_idx.reshape(nsteps, _E, 1)
    sel_d2 = sel_d2.reshape(nsteps, _E, 1)
    mul_e = mul_e.reshape(nsteps, _E, 1)
    bias_e = bias_e.reshape(nsteps, _E, 1)

    out = pl.pallas_call(
        _edge_body,
        grid=(nsteps,),
        in_specs=[
            pl.BlockSpec((1, _E, 1), lambda g: (g, 0, 0)),
            pl.BlockSpec((1, _E, 1), lambda g: (g, 0, 0)),
            pl.BlockSpec((1, _E, 1), lambda g: (g, 0, 0)),
            pl.BlockSpec((1, _E, 1), lambda g: (g, 0, 0)),
            pl.BlockSpec((1, K), lambda g: (0, 0)),
            pl.BlockSpec((1, K), lambda g: (0, 0)),
            pl.BlockSpec((K, K), lambda g: (0, 0)),
            pl.BlockSpec((1, K), lambda g: (0, 0)),
            pl.BlockSpec((K, H), lambda g: (0, 0)),
            pl.BlockSpec((1, H), lambda g: (0, 0)),
        ],
        out_specs=pl.BlockSpec(
            (1, H, _BI, T), lambda g: (g // steps_per_b, 0, g % steps_per_b, 0)),
        out_shape=jax.ShapeDtypeStruct((B, H, T, T), jnp.float32),
    )(sel_idx, sel_d2, mul_e, bias_e,
      means.reshape(1, K), stds.reshape(1, K), W1, b1.reshape(1, K),
      W2, b2.reshape(1, H))
    return out


# speedup vs baseline: 11.8437x; 1.0164x over previous
import functools
import math

import jax
import jax.numpy as jnp
import numpy as np
from jax import lax
from jax.experimental import pallas as pl
from jax.experimental.pallas import tpu as pltpu
from jax.experimental.pallas import tpu_sc as plsc

B, T, NUM_TYPES, K, H = 4, 256, 128, 128, 32
RADIUS, MAX_NBR, MAX_REP = 5.0, 32, 2
C = (2 * MAX_REP + 1) * (2 * MAX_REP + 1)
NCAND = T * C
_SQRT_2PI = math.sqrt(2.0 * math.pi)

_RG = 8
_BI = 64
_E = _BI * MAX_NBR


_L = 16
_NW = 32
_RPW = (B * T) // _NW
_NCH = NCAND // _L
_R2 = RADIUS * RADIUS


def _sc_select_body(p2x_hbm, p2y_hbm, p2z_hbm, pxs_hbm, pys_hbm, pzs_hbm,
                    nav_hbm, at_hbm, mul_hbm, bias_hbm,
                    outk_hbm, outd_hbm, outm_hbm, outb_hbm,
                    p2x, p2y, p2z, pxs, pys, pzs, nav_v,
                    atb_vm, mul_vm, bias_vm,
                    cd2, ck, cmin, outk_v, outd_v, outm_v, outb_v):
    cid = lax.axis_index("c")
    sid = lax.axis_index("s")
    wid = sid * 2 + cid
    base = wid * _RPW
    b = base // T

    pltpu.sync_copy(p2x_hbm.at[b], p2x)
    pltpu.sync_copy(p2y_hbm.at[b], p2y)
    pltpu.sync_copy(p2z_hbm.at[b], p2z)
    pltpu.sync_copy(pxs_hbm.at[pl.ds(base, _RPW)], pxs)
    pltpu.sync_copy(pys_hbm.at[pl.ds(base, _RPW)], pys)
    pltpu.sync_copy(pzs_hbm.at[pl.ds(base, _RPW)], pzs)
    pltpu.sync_copy(nav_hbm, nav_v)
    pltpu.sync_copy(at_hbm.at[b], atb_vm)
    pltpu.sync_copy(mul_hbm, mul_vm)
    pltpu.sync_copy(bias_hbm, bias_vm)

    iota = lax.broadcasted_iota(jnp.int32, (_L,), 0)
    inf16 = jnp.full((_L,), jnp.inf, jnp.float32)
    nch_scan = nav_v[pl.ds(0, _L)][0] * (T // _L)

    def row_body(r, _):
        px = pxs[r]
        py = pys[r]
        pz = pzs[r]

        def ch_body(ci, cnt):
            off = ci * _L
            dx = p2x[pl.ds(off, _L)] - px
            dy = p2y[pl.ds(off, _L)] - py
            dz = p2z[pl.ds(off, _L)] - pz
            d2 = dx * dx + dy * dy + dz * dz
            m = (d2 <= _R2) & (d2 > 1e-4)
            plsc.store_compressed(cd2.at[pl.ds(cnt, _L)], d2, mask=m)
            plsc.store_compressed(ck.at[pl.ds(cnt, _L)], (off + iota) & (T - 1),
                                  mask=m)
            return cnt + jnp.sum(m.astype(jnp.int32))

        cnt = lax.fori_loop(0, nch_scan, ch_body, jnp.int32(0))
        cd2[pl.ds(cnt, _L)] = inf16

        nch_c = (cnt + _L - 1) // _L
        lane0 = iota == 0

        def cm_body(ci, _):
            mv = jnp.min(cd2[pl.ds(ci * _L, _L)])
            plsc.store_compressed(cmin.at[pl.ds(ci, _L)],
                                  jnp.full((_L,), mv, jnp.float32), mask=lane0)
            return 0

        lax.fori_loop(0, nch_c, cm_body, 0)
        cmin[pl.ds(nch_c, _L)] = inf16
        ncm = (nch_c + _L - 1) // _L

        def ex_body(n, carry):
            kacc, dacc = carry

            def gm_body(ci, gcarry):
                bv, bi = gcarry
                v = cmin[pl.ds(ci * _L, _L)]
                mv = jnp.min(v)
                lane = jnp.min(plsc.all_reduce_ffs(v == mv))
                better = mv < bv
                return (jnp.where(better, mv, bv),
                        jnp.where(better, ci * _L + lane, bi))

            gmin, fch = lax.fori_loop(0, ncm, gm_body,
                                      (jnp.float32(jnp.inf), jnp.int32(0)))
            coff = fch * _L
            v = cd2[pl.ds(coff, _L)]
            lane = jnp.min(plsc.all_reduce_ffs(v == gmin))
            vk = ck[pl.ds(coff, _L)]
            kval = jnp.min(jnp.where(iota == lane, vk, jnp.int32(2 ** 30)))
            kacc = jnp.where(iota == (n % _L), jnp.full((_L,), kval, jnp.int32), kacc)
            dacc = jnp.where(iota == (n % _L), jnp.full((_L,), gmin, jnp.float32), dacc)
            v2 = jnp.where(iota == lane, jnp.inf, v)
            cd2[pl.ds(coff, _L)] = v2
            plsc.store_compressed(cmin.at[pl.ds(fch, _L)],
                                  jnp.full((_L,), jnp.min(v2), jnp.float32),
                                  mask=lane0)
            return kacc, dacc

        lrow = base - b * T + r
        a_i = plsc.load_gather(atb_vm, [jnp.full((_L,), lrow, jnp.int32)])

        def emit_half(h0, kacc, dacc):
            outk_v[r, pl.ds(h0, _L)] = kacc
            outd_v[r, pl.ds(h0, _L)] = dacc
            a_j = plsc.load_gather(atb_vm, [kacc & (T - 1)])
            etype = a_i * NUM_TYPES + a_j
            outm_v[r, pl.ds(h0, _L)] = plsc.load_gather(mul_vm, [etype])
            outb_v[r, pl.ds(h0, _L)] = plsc.load_gather(bias_vm, [etype])

        z16i = jnp.zeros((_L,), jnp.int32)
        z16f = jnp.zeros((_L,), jnp.float32)
        kacc, dacc = lax.fori_loop(0, _L, ex_body, (z16i, z16f))
        emit_half(0, kacc, dacc)
        kacc, dacc = lax.fori_loop(_L, MAX_NBR, ex_body, (z16i, z16f))
        emit_half(_L, kacc, dacc)
        return 0

    lax.fori_loop(0, _RPW, row_body, 0)
    pltpu.sync_copy(outk_v, outk_hbm.at[pl.ds(base, _RPW)])
    pltpu.sync_copy(outd_v, outd_hbm.at[pl.ds(base, _RPW)])
    pltpu.sync_copy(outm_v, outm_hbm.at[pl.ds(base, _RPW)])
    pltpu.sync_copy(outb_v, outb_hbm.at[pl.ds(base, _RPW)])


_sc_select = functools.partial(
    pl.kernel,
    out_type=[jax.ShapeDtypeStruct((B * T, MAX_NBR), jnp.int32),
              jax.ShapeDtypeStruct((B * T, MAX_NBR), jnp.float32),
              jax.ShapeDtypeStruct((B * T, MAX_NBR), jnp.float32),
              jax.ShapeDtypeStruct((B * T, MAX_NBR), jnp.float32)],
    mesh=plsc.VectorSubcoreMesh(core_axis_name="c", subcore_axis_name="s"),
    scratch_types=[
        pltpu.VMEM((NCAND,), jnp.float32),
        pltpu.VMEM((NCAND,), jnp.float32),
        pltpu.VMEM((NCAND,), jnp.float32),
        pltpu.VMEM((_RPW, _L), jnp.float32),
        pltpu.VMEM((_RPW, _L), jnp.float32),
        pltpu.VMEM((_RPW, _L), jnp.float32),
        pltpu.VMEM((_L,), jnp.int32),
        pltpu.VMEM((T,), jnp.int32),
        pltpu.VMEM((NUM_TYPES * NUM_TYPES,), jnp.float32),
        pltpu.VMEM((NUM_TYPES * NUM_TYPES,), jnp.float32),
        pltpu.VMEM((NCAND + _L,), jnp.float32),
        pltpu.VMEM((NCAND + _L,), jnp.int32),
        pltpu.VMEM((_NCH + _L,), jnp.float32),
        pltpu.VMEM((_RPW, MAX_NBR), jnp.int32),
        pltpu.VMEM((_RPW, MAX_NBR), jnp.float32),
        pltpu.VMEM((_RPW, MAX_NBR), jnp.float32),
        pltpu.VMEM((_RPW, MAX_NBR), jnp.float32),
    ],
    compiler_params=pltpu.CompilerParams(needs_layout_passes=False),
)(_sc_select_body)


def _select_body(p2x_ref, p2y_ref, p2z_ref, px_ref, py_ref, pz_ref,
                 idx_ref, d2_ref):
    p2x = p2x_ref[0, 0, :]
    p2y = p2y_ref[0, 0, :]
    p2z = p2z_ref[0, 0, :]
    px = px_ref[0]
    py = py_ref[0]
    pz = pz_ref[0]
    dx = p2x - px
    dy = p2y - py
    dz = p2z - pz
    d2 = dx * dx + dy * dy + dz * dz
    ok = (d2 <= RADIUS * RADIUS) & (d2 > 1e-4)
    d2m = jnp.where(ok, d2, jnp.inf)
    iota = jax.lax.broadcasted_iota(jnp.int32, (_RG, NCAND), 1)
    idx_cols = []
    d2_cols = []
    for _ in range(MAX_NBR):
        rowmin = jnp.min(d2m, axis=1, keepdims=True)
        cand = jnp.where(d2m == rowmin, iota, jnp.int32(2 ** 30))
        amin = jnp.min(cand, axis=1, keepdims=True)
        idx_cols.append(amin)
        d2_cols.append(rowmin)
        d2m = jnp.where(iota == amin, jnp.inf, d2m)
    idx_ref[0] = jnp.concatenate(idx_cols, axis=1)
    d2_ref[0] = jnp.concatenate(d2_cols, axis=1)


def _edge_body(idx_ref, d2_ref, mule_ref, biase_ref,
               means_ref, stds_ref, w1_ref, b1_ref, w2_ref, b2_ref, out_ref):
    k = idx_ref[0]
    d2s = d2_ref[0]
    valid = d2s <= RADIUS * RADIUS
    j = k.astype(jnp.float32)

    iota_t = jax.lax.broadcasted_iota(jnp.int32, (_E, T), 1).astype(jnp.float32)
    oj = (j == iota_t).astype(jnp.float32)
    mul_e = mule_ref[0]
    bias_e = biase_ref[0]

    d2c = jnp.where(valid, d2s, 0.0)
    dist = jnp.sqrt(d2c + 1e-12)
    x = mul_e * dist + bias_e

    std = jnp.abs(stds_ref[...]) + 1e-5
    pre = (x - means_ref[...]) / std
    g = jnp.exp(-0.5 * pre * pre) / (_SQRT_2PI * std)
    h1 = jax.nn.gelu(jnp.dot(g, w1_ref[...], preferred_element_type=jnp.float32, precision=jax.lax.Precision.HIGHEST)
                     + b1_ref[...])
    h = jnp.dot(h1, w2_ref[...], preferred_element_type=jnp.float32, precision=jax.lax.Precision.HIGHEST) + b2_ref[...]
    h = jnp.where(valid, h, 0.0)
    hb = h.astype(jnp.bfloat16).astype(jnp.float32)
    hr = h - hb
    htb = hb.T
    htr = hr.T

    for i in range(_BI):
        sl = slice(i * MAX_NBR, (i + 1) * MAX_NBR)
        si = oj[sl, :]
        out_ref[0, :, i, :] = (
            jnp.dot(htb[:, sl], si, preferred_element_type=jnp.float32)
            + jnp.dot(htr[:, sl], si, preferred_element_type=jnp.float32))


def kernel(pos, atom_types, natoms, cell, means, stds, mul_w, bias_w,
           W1, b1, W2, b2):
    del natoms
    cross_a2a3 = jnp.cross(cell[:, 1], cell[:, 2])
    cell_vol = jnp.sum(cell[:, 0] * cross_a2a3, axis=-1, keepdims=True)
    crosses = [cross_a2a3, jnp.cross(cell[:, 2], cell[:, 0]),
               jnp.cross(cell[:, 0], cell[:, 1])]
    pbc = (True, True, False)
    rep_caps = []
    for dim in range(3):
        if pbc[dim]:
            inv = jnp.linalg.norm(crosses[dim] / cell_vol, axis=-1)
            rep_caps.append(jnp.minimum(jnp.max(jnp.ceil(RADIUS * inv)), float(MAX_REP)))
        else:
            rep_caps.append(jnp.asarray(0.0, jnp.float32))
    static_reps = [MAX_REP if pbc[dim] else 0 for dim in range(3)]
    cells_per_dim = [np.arange(-r, r + 1, dtype=np.float32) for r in static_reps]
    mg = np.meshgrid(*cells_per_dim, indexing="ij")
    unit_cell = jnp.asarray(np.stack([m.reshape(-1) for m in mg], axis=1))
    cell_ok = jnp.ones((C,), bool)
    for dim in range(3):
        cell_ok = cell_ok & (jnp.abs(unit_cell[:, dim]) <= rep_caps[dim])
    pbc_off = jnp.einsum("ci,bij->bcj", unit_cell, cell)
    order = jnp.argsort(jnp.logical_not(cell_ok).astype(jnp.int32), stable=True)
    n_alive = jnp.sum(cell_ok.astype(jnp.int32))
    pbc_off = pbc_off[:, order, :]
    pos2 = pbc_off[:, :, None, :] + pos[:, None, :, :]
    pos2 = pos2.reshape(B, NCAND, 3)

    p2x = pos2[..., 0]
    p2y = pos2[..., 1]
    p2z = pos2[..., 2]
    pos_flat = pos.reshape(B * T, 3)
    pxs = jnp.broadcast_to(pos_flat[:, 0:1], (B * T, _L))
    pys = jnp.broadcast_to(pos_flat[:, 1:2], (B * T, _L))
    pzs = jnp.broadcast_to(pos_flat[:, 2:3], (B * T, _L))
    nav = jnp.full((_L,), n_alive, jnp.int32)

    sel_idx, sel_d2, mul_e, bias_e = _sc_select(
        p2x, p2y, p2z, pxs, pys, pzs, nav, atom_types.astype(jnp.int32),
        mul_w, bias_w)
    nsteps = B * T // _BI
    steps_per_b = T // _BI
    sel_idx = sel_idx.reshape(nsteps, _E, 1)
    sel_d2 = sel_d2.reshape(nsteps, _E, 1)
    mul_e = mul_e.reshape(nsteps, _E, 1)
    bias_e = bias_e.reshape(nsteps, _E, 1)

    out = pl.pallas_call(
        _edge_body,
        grid=(nsteps,),
        in_specs=[
            pl.BlockSpec((1, _E, 1), lambda g: (g, 0, 0)),
            pl.BlockSpec((1, _E, 1), lambda g: (g, 0, 0)),
            pl.BlockSpec((1, _E, 1), lambda g: (g, 0, 0)),
            pl.BlockSpec((1, _E, 1), lambda g: (g, 0, 0)),
            pl.BlockSpec((1, K), lambda g: (0, 0)),
            pl.BlockSpec((1, K), lambda g: (0, 0)),
            pl.BlockSpec((K, K), lambda g: (0, 0)),
            pl.BlockSpec((1, K), lambda g: (0, 0)),
            pl.BlockSpec((K, H), lambda g: (0, 0)),
            pl.BlockSpec((1, H), lambda g: (0, 0)),
        ],
        out_specs=pl.BlockSpec(
            (1, H, _BI, T), lambda g: (g // steps_per_b, 0, g % steps_per_b, 0)),
        out_shape=jax.ShapeDtypeStruct((B, H, T, T), jnp.float32),
    )(sel_idx, sel_d2, mul_e, bias_e,
      means.reshape(1, K), stds.reshape(1, K), W1, b1.reshape(1, K),
      W2, b2.reshape(1, H))
    return out


# speedup vs baseline: 14.0548x; 1.1867x over previous
import functools
import math

import jax
import jax.numpy as jnp
import numpy as np
from jax import lax
from jax.experimental import pallas as pl
from jax.experimental.pallas import tpu as pltpu
from jax.experimental.pallas import tpu_sc as plsc

B, T, NUM_TYPES, K, H = 4, 256, 128, 128, 32
RADIUS, MAX_NBR, MAX_REP = 5.0, 32, 2
C = (2 * MAX_REP + 1) * (2 * MAX_REP + 1)
NCAND = T * C
_SQRT_2PI = math.sqrt(2.0 * math.pi)

_RG = 8
_BI = 64
_E = _BI * MAX_NBR


_L = 16
_NW = 32
_RPW = (B * T) // _NW
_NCH = NCAND // _L
_R2 = RADIUS * RADIUS


def _sc_select_body(p2x_hbm, p2y_hbm, p2z_hbm, pxs_hbm, pys_hbm, pzs_hbm,
                    nav_hbm, at_hbm, mul_hbm, bias_hbm,
                    outk_hbm, outd_hbm, outm_hbm, outb_hbm,
                    p2x, p2y, p2z, pxs, pys, pzs, nav_v,
                    atb_vm, mul_vm, bias_vm,
                    cd2, ck, cmin, cd2b, ckb, cminb,
                    outk_v, outd_v, outm_v, outb_v):
    cid = lax.axis_index("c")
    sid = lax.axis_index("s")
    wid = sid * 2 + cid
    base = wid * _RPW
    b = base // T

    pltpu.sync_copy(p2x_hbm.at[b], p2x)
    pltpu.sync_copy(p2y_hbm.at[b], p2y)
    pltpu.sync_copy(p2z_hbm.at[b], p2z)
    pltpu.sync_copy(pxs_hbm.at[pl.ds(base, _RPW)], pxs)
    pltpu.sync_copy(pys_hbm.at[pl.ds(base, _RPW)], pys)
    pltpu.sync_copy(pzs_hbm.at[pl.ds(base, _RPW)], pzs)
    pltpu.sync_copy(nav_hbm, nav_v)
    pltpu.sync_copy(at_hbm.at[b], atb_vm)
    pltpu.sync_copy(mul_hbm, mul_vm)
    pltpu.sync_copy(bias_hbm, bias_vm)

    iota = lax.broadcasted_iota(jnp.int32, (_L,), 0)
    inf16 = jnp.full((_L,), jnp.inf, jnp.float32)
    lane0 = iota == 0
    nch_scan = nav_v[pl.ds(0, _L)][0] * (T // _L)

    def pair_body(rp, _):
        ra = rp * 2
        rb = ra + 1
        pxa = pxs[ra]
        pya = pys[ra]
        pza = pzs[ra]
        pxb = pxs[rb]
        pyb = pys[rb]
        pzb = pzs[rb]

        def ch_body(ci, carry):
            cnta, cntb = carry
            off = ci * _L
            vx = p2x[pl.ds(off, _L)]
            vy = p2y[pl.ds(off, _L)]
            vz = p2z[pl.ds(off, _L)]
            jvec = (off + iota) & (T - 1)
            dxa = vx - pxa
            dya = vy - pya
            dza = vz - pza
            d2a = dxa * dxa + dya * dya + dza * dza
            ma = (d2a <= _R2) & (d2a > 1e-4)
            dxb = vx - pxb
            dyb = vy - pyb
            dzb = vz - pzb
            d2b = dxb * dxb + dyb * dyb + dzb * dzb
            mb = (d2b <= _R2) & (d2b > 1e-4)
            plsc.store_compressed(cd2.at[pl.ds(cnta, _L)], d2a, mask=ma)
            plsc.store_compressed(ck.at[pl.ds(cnta, _L)], jvec, mask=ma)
            plsc.store_compressed(cd2b.at[pl.ds(cntb, _L)], d2b, mask=mb)
            plsc.store_compressed(ckb.at[pl.ds(cntb, _L)], jvec, mask=mb)
            return (cnta + jnp.sum(ma.astype(jnp.int32)),
                    cntb + jnp.sum(mb.astype(jnp.int32)))

        cnta, cntb = lax.fori_loop(0, nch_scan, ch_body,
                                   (jnp.int32(0), jnp.int32(0)))
        cd2[pl.ds(cnta, _L)] = inf16
        cd2b[pl.ds(cntb, _L)] = inf16

        nch_ca = (cnta + _L - 1) // _L
        nch_cb = (cntb + _L - 1) // _L
        nch_mx = jnp.maximum(nch_ca, nch_cb)

        def cm_body(ci, _):
            mva = jnp.min(cd2[pl.ds(ci * _L, _L)])
            mvb = jnp.min(cd2b[pl.ds(ci * _L, _L)])
            mva = jnp.where(ci < nch_ca, mva, jnp.inf)
            mvb = jnp.where(ci < nch_cb, mvb, jnp.inf)
            plsc.store_compressed(cmin.at[pl.ds(ci, _L)],
                                  jnp.full((_L,), mva, jnp.float32), mask=lane0)
            plsc.store_compressed(cminb.at[pl.ds(ci, _L)],
                                  jnp.full((_L,), mvb, jnp.float32), mask=lane0)
            return 0

        lax.fori_loop(0, nch_mx, cm_body, 0)
        cmin[pl.ds(nch_mx, _L)] = inf16
        cminb[pl.ds(nch_mx, _L)] = inf16
        ncm = (nch_mx + _L - 1) // _L

        def ex_body(n, carry):
            kacca, dacca, kaccb, daccb = carry

            def gm_body(ci, gcarry):
                bva, bia, bvb, bib = gcarry
                va = cmin[pl.ds(ci * _L, _L)]
                vb = cminb[pl.ds(ci * _L, _L)]
                mva = jnp.min(va)
                mvb = jnp.min(vb)
                lanea = jnp.min(plsc.all_reduce_ffs(va == mva))
                laneb = jnp.min(plsc.all_reduce_ffs(vb == mvb))
                bettera = mva < bva
                betterb = mvb < bvb
                return (jnp.where(bettera, mva, bva),
                        jnp.where(bettera, ci * _L + lanea, bia),
                        jnp.where(betterb, mvb, bvb),
                        jnp.where(betterb, ci * _L + laneb, bib))

            gmina, fcha, gminb, fchb = lax.fori_loop(
                0, ncm, gm_body,
                (jnp.float32(jnp.inf), jnp.int32(0),
                 jnp.float32(jnp.inf), jnp.int32(0)))
            coffa = fcha * _L
            coffb = fchb * _L
            va = cd2[pl.ds(coffa, _L)]
            vb = cd2b[pl.ds(coffb, _L)]
            lanea = jnp.min(plsc.all_reduce_ffs(va == gmina))
            laneb = jnp.min(plsc.all_reduce_ffs(vb == gminb))
            vka = ck[pl.ds(coffa, _L)]
            vkb = ckb[pl.ds(coffb, _L)]
            kvala = jnp.min(jnp.where(iota == lanea, vka, jnp.int32(2 ** 30)))
            kvalb = jnp.min(jnp.where(iota == laneb, vkb, jnp.int32(2 ** 30)))
            sel = iota == (n % _L)
            kacca = jnp.where(sel, jnp.full((_L,), kvala, jnp.int32), kacca)
            dacca = jnp.where(sel, jnp.full((_L,), gmina, jnp.float32), dacca)
            kaccb = jnp.where(sel, jnp.full((_L,), kvalb, jnp.int32), kaccb)
            daccb = jnp.where(sel, jnp.full((_L,), gminb, jnp.float32), daccb)
            v2a = jnp.where(iota == lanea, jnp.inf, va)
            v2b = jnp.where(iota == laneb, jnp.inf, vb)
            cd2[pl.ds(coffa, _L)] = v2a
            cd2b[pl.ds(coffb, _L)] = v2b
            plsc.store_compressed(cmin.at[pl.ds(fcha, _L)],
                                  jnp.full((_L,), jnp.min(v2a), jnp.float32),
                                  mask=lane0)
            plsc.store_compressed(cminb.at[pl.ds(fchb, _L)],
                                  jnp.full((_L,), jnp.min(v2b), jnp.float32),
                                  mask=lane0)
            return kacca, dacca, kaccb, daccb

        lrow = base - b * T + ra
        a_ia = plsc.load_gather(atb_vm, [jnp.full((_L,), lrow, jnp.int32)])
        a_ib = plsc.load_gather(atb_vm, [jnp.full((_L,), lrow + 1, jnp.int32)])

        def emit_half(h0, r, a_i, kacc, dacc):
            outk_v[r, pl.ds(h0, _L)] = kacc
            outd_v[r, pl.ds(h0, _L)] = dacc
            a_j = plsc.load_gather(atb_vm, [kacc & (T - 1)])
            etype = a_i * NUM_TYPES + a_j
            outm_v[r, pl.ds(h0, _L)] = plsc.load_gather(mul_vm, [etype])
            outb_v[r, pl.ds(h0, _L)] = plsc.load_gather(bias_vm, [etype])

        z16i = jnp.zeros((_L,), jnp.int32)
        z16f = jnp.zeros((_L,), jnp.float32)
        acc = lax.fori_loop(0, _L, ex_body, (z16i, z16f, z16i, z16f))
        emit_half(0, ra, a_ia, acc[0], acc[1])
        emit_half(0, rb, a_ib, acc[2], acc[3])
        acc = lax.fori_loop(_L, MAX_NBR, ex_body, (z16i, z16f, z16i, z16f))
        emit_half(_L, ra, a_ia, acc[0], acc[1])
        emit_half(_L, rb, a_ib, acc[2], acc[3])
        return 0

    lax.fori_loop(0, _RPW // 2, pair_body, 0)
    pltpu.sync_copy(outk_v, outk_hbm.at[pl.ds(base, _RPW)])
    pltpu.sync_copy(outd_v, outd_hbm.at[pl.ds(base, _RPW)])
    pltpu.sync_copy(outm_v, outm_hbm.at[pl.ds(base, _RPW)])
    pltpu.sync_copy(outb_v, outb_hbm.at[pl.ds(base, _RPW)])


_sc_select = functools.partial(
    pl.kernel,
    out_type=[jax.ShapeDtypeStruct((B * T, MAX_NBR), jnp.int32),
              jax.ShapeDtypeStruct((B * T, MAX_NBR), jnp.float32),
              jax.ShapeDtypeStruct((B * T, MAX_NBR), jnp.float32),
              jax.ShapeDtypeStruct((B * T, MAX_NBR), jnp.float32)],
    mesh=plsc.VectorSubcoreMesh(core_axis_name="c", subcore_axis_name="s"),
    scratch_types=[
        pltpu.VMEM((NCAND,), jnp.float32),
        pltpu.VMEM((NCAND,), jnp.float32),
        pltpu.VMEM((NCAND,), jnp.float32),
        pltpu.VMEM((_RPW, _L), jnp.float32),
        pltpu.VMEM((_RPW, _L), jnp.float32),
        pltpu.VMEM((_RPW, _L), jnp.float32),
        pltpu.VMEM((_L,), jnp.int32),
        pltpu.VMEM((T,), jnp.int32),
        pltpu.VMEM((NUM_TYPES * NUM_TYPES,), jnp.float32),
        pltpu.VMEM((NUM_TYPES * NUM_TYPES,), jnp.float32),
        pltpu.VMEM((NCAND + _L,), jnp.float32),
        pltpu.VMEM((NCAND + _L,), jnp.int32),
        pltpu.VMEM((_NCH + _L,), jnp.float32),
        pltpu.VMEM((NCAND + _L,), jnp.float32),
        pltpu.VMEM((NCAND + _L,), jnp.int32),
        pltpu.VMEM((_NCH + _L,), jnp.float32),
        pltpu.VMEM((_RPW, MAX_NBR), jnp.int32),
        pltpu.VMEM((_RPW, MAX_NBR), jnp.float32),
        pltpu.VMEM((_RPW, MAX_NBR), jnp.float32),
        pltpu.VMEM((_RPW, MAX_NBR), jnp.float32),
    ],
    compiler_params=pltpu.CompilerParams(needs_layout_passes=False),
)(_sc_select_body)


def _select_body(p2x_ref, p2y_ref, p2z_ref, px_ref, py_ref, pz_ref,
                 idx_ref, d2_ref):
    p2x = p2x_ref[0, 0, :]
    p2y = p2y_ref[0, 0, :]
    p2z = p2z_ref[0, 0, :]
    px = px_ref[0]
    py = py_ref[0]
    pz = pz_ref[0]
    dx = p2x - px
    dy = p2y - py
    dz = p2z - pz
    d2 = dx * dx + dy * dy + dz * dz
    ok = (d2 <= RADIUS * RADIUS) & (d2 > 1e-4)
    d2m = jnp.where(ok, d2, jnp.inf)
    iota = jax.lax.broadcasted_iota(jnp.int32, (_RG, NCAND), 1)
    idx_cols = []
    d2_cols = []
    for _ in range(MAX_NBR):
        rowmin = jnp.min(d2m, axis=1, keepdims=True)
        cand = jnp.where(d2m == rowmin, iota, jnp.int32(2 ** 30))
        amin = jnp.min(cand, axis=1, keepdims=True)
        idx_cols.append(amin)
        d2_cols.append(rowmin)
        d2m = jnp.where(iota == amin, jnp.inf, d2m)
    idx_ref[0] = jnp.concatenate(idx_cols, axis=1)
    d2_ref[0] = jnp.concatenate(d2_cols, axis=1)


def _edge_body(idx_ref, d2_ref, mule_ref, biase_ref,
               means_ref, stds_ref, w1_ref, b1_ref, w2_ref, b2_ref, out_ref):
    k = idx_ref[0]
    d2s = d2_ref[0]
    valid = d2s <= RADIUS * RADIUS
    j = k.astype(jnp.float32)

    iota_t = jax.lax.broadcasted_iota(jnp.int32, (_E, T), 1).astype(jnp.float32)
    oj = (j == iota_t).astype(jnp.float32)
    mul_e = mule_ref[0]
    bias_e = biase_ref[0]

    d2c = jnp.where(valid, d2s, 0.0)
    dist = jnp.sqrt(d2c + 1e-12)
    x = mul_e * dist + bias_e

    std = jnp.abs(stds_ref[...]) + 1e-5
    pre = (x - means_ref[...]) / std
    g = jnp.exp(-0.5 * pre * pre) / (_SQRT_2PI * std)
    h1 = jax.nn.gelu(jnp.dot(g, w1_ref[...], preferred_element_type=jnp.float32, precision=jax.lax.Precision.HIGHEST)
                     + b1_ref[...])
    h = jnp.dot(h1, w2_ref[...], preferred_element_type=jnp.float32, precision=jax.lax.Precision.HIGHEST) + b2_ref[...]
    h = jnp.where(valid, h, 0.0)
    hb = h.astype(jnp.bfloat16).astype(jnp.float32)
    hr = h - hb
    htb = hb.T
    htr = hr.T

    for i in range(_BI):
        sl = slice(i * MAX_NBR, (i + 1) * MAX_NBR)
        si = oj[sl, :]
        out_ref[0, :, i, :] = (
            jnp.dot(htb[:, sl], si, preferred_element_type=jnp.float32)
            + jnp.dot(htr[:, sl], si, preferred_element_type=jnp.float32))


def kernel(pos, atom_types, natoms, cell, means, stds, mul_w, bias_w,
           W1, b1, W2, b2):
    del natoms
    cross_a2a3 = jnp.cross(cell[:, 1], cell[:, 2])
    cell_vol = jnp.sum(cell[:, 0] * cross_a2a3, axis=-1, keepdims=True)
    crosses = [cross_a2a3, jnp.cross(cell[:, 2], cell[:, 0]),
               jnp.cross(cell[:, 0], cell[:, 1])]
    pbc = (True, True, False)
    rep_caps = []
    for dim in range(3):
        if pbc[dim]:
            inv = jnp.linalg.norm(crosses[dim] / cell_vol, axis=-1)
            rep_caps.append(jnp.minimum(jnp.max(jnp.ceil(RADIUS * inv)), float(MAX_REP)))
        else:
            rep_caps.append(jnp.asarray(0.0, jnp.float32))
    static_reps = [MAX_REP if pbc[dim] else 0 for dim in range(3)]
    cells_per_dim = [np.arange(-r, r + 1, dtype=np.float32) for r in static_reps]
    mg = np.meshgrid(*cells_per_dim, indexing="ij")
    unit_cell = jnp.asarray(np.stack([m.reshape(-1) for m in mg], axis=1))
    cell_ok = jnp.ones((C,), bool)
    for dim in range(3):
        cell_ok = cell_ok & (jnp.abs(unit_cell[:, dim]) <= rep_caps[dim])
    pbc_off = jnp.einsum("ci,bij->bcj", unit_cell, cell)
    order = jnp.argsort(jnp.logical_not(cell_ok).astype(jnp.int32), stable=True)
    n_alive = jnp.sum(cell_ok.astype(jnp.int32))
    pbc_off = pbc_off[:, order, :]
    pos2 = pbc_off[:, :, None, :] + pos[:, None, :, :]
    pos2 = pos2.reshape(B, NCAND, 3)

    p2x = pos2[..., 0]
    p2y = pos2[..., 1]
    p2z = pos2[..., 2]
    pos_flat = pos.reshape(B * T, 3)
    pxs = jnp.broadcast_to(pos_flat[:, 0:1], (B * T, _L))
    pys = jnp.broadcast_to(pos_flat[:, 1:2], (B * T, _L))
    pzs = jnp.broadcast_to(pos_flat[:, 2:3], (B * T, _L))
    nav = jnp.full((_L,), n_alive, jnp.int32)

    sel_idx, sel_d2, mul_e, bias_e = _sc_select(
        p2x, p2y, p2z, pxs, pys, pzs, nav, atom_types.astype(jnp.int32),
        mul_w, bias_w)
    nsteps = B * T // _BI
    steps_per_b = T // _BI
    sel_idx = sel_idx.reshape(nsteps, _E, 1)
    sel_d2 = sel_d2.reshape(nsteps, _E, 1)
    mul_e = mul_e.reshape(nsteps, _E, 1)
    bias_e = bias_e.reshape(nsteps, _E, 1)

    out = pl.pallas_call(
        _edge_body,
        grid=(nsteps,),
        in_specs=[
            pl.BlockSpec((1, _E, 1), lambda g: (g, 0, 0)),
            pl.BlockSpec((1, _E, 1), lambda g: (g, 0, 0)),
            pl.BlockSpec((1, _E, 1), lambda g: (g, 0, 0)),
            pl.BlockSpec((1, _E, 1), lambda g: (g, 0, 0)),
            pl.BlockSpec((1, K), lambda g: (0, 0)),
            pl.BlockSpec((1, K), lambda g: (0, 0)),
            pl.BlockSpec((K, K), lambda g: (0, 0)),
            pl.BlockSpec((1, K), lambda g: (0, 0)),
            pl.BlockSpec((K, H), lambda g: (0, 0)),
            pl.BlockSpec((1, H), lambda g: (0, 0)),
        ],
        out_specs=pl.BlockSpec(
            (1, H, _BI, T), lambda g: (g // steps_per_b, 0, g % steps_per_b, 0)),
        out_shape=jax.ShapeDtypeStruct((B, H, T, T), jnp.float32),
    )(sel_idx, sel_d2, mul_e, bias_e,
      means.reshape(1, K), stds.reshape(1, K), W1, b1.reshape(1, K),
      W2, b2.reshape(1, H))
    return out


# speedup vs baseline: 14.1220x; 1.0048x over previous
import functools
import math

import jax
import jax.numpy as jnp
import numpy as np
from jax import lax
from jax.experimental import pallas as pl
from jax.experimental.pallas import tpu as pltpu
from jax.experimental.pallas import tpu_sc as plsc

B, T, NUM_TYPES, K, H = 4, 256, 128, 128, 32
RADIUS, MAX_NBR, MAX_REP = 5.0, 32, 2
C = (2 * MAX_REP + 1) * (2 * MAX_REP + 1)
NCAND = T * C
_SQRT_2PI = math.sqrt(2.0 * math.pi)

_RG = 8
_BI = 128
_E = _BI * MAX_NBR


_L = 16
_NW = 32
_RPW = (B * T) // _NW
_NCH = NCAND // _L
_R2 = RADIUS * RADIUS


def _sc_select_body(p2x_hbm, p2y_hbm, p2z_hbm, pxs_hbm, pys_hbm, pzs_hbm,
                    nav_hbm, at_hbm, mul_hbm, bias_hbm,
                    outk_hbm, outd_hbm, outm_hbm, outb_hbm,
                    p2x, p2y, p2z, pxs, pys, pzs, nav_v,
                    atb_vm, mul_vm, bias_vm,
                    cd2, ck, cmin, cd2b, ckb, cminb,
                    outk_v, outd_v, outm_v, outb_v):
    cid = lax.axis_index("c")
    sid = lax.axis_index("s")
    wid = sid * 2 + cid
    base = wid * _RPW
    b = base // T

    pltpu.sync_copy(p2x_hbm.at[b], p2x)
    pltpu.sync_copy(p2y_hbm.at[b], p2y)
    pltpu.sync_copy(p2z_hbm.at[b], p2z)
    pltpu.sync_copy(pxs_hbm.at[pl.ds(base, _RPW)], pxs)
    pltpu.sync_copy(pys_hbm.at[pl.ds(base, _RPW)], pys)
    pltpu.sync_copy(pzs_hbm.at[pl.ds(base, _RPW)], pzs)
    pltpu.sync_copy(nav_hbm, nav_v)
    pltpu.sync_copy(at_hbm.at[b], atb_vm)
    pltpu.sync_copy(mul_hbm, mul_vm)
    pltpu.sync_copy(bias_hbm, bias_vm)

    iota = lax.broadcasted_iota(jnp.int32, (_L,), 0)
    inf16 = jnp.full((_L,), jnp.inf, jnp.float32)
    lane0 = iota == 0
    nch_scan = nav_v[pl.ds(0, _L)][0] * (T // _L)

    def pair_body(rp, _):
        ra = rp * 2
        rb = ra + 1
        pxa = pxs[ra]
        pya = pys[ra]
        pza = pzs[ra]
        pxb = pxs[rb]
        pyb = pys[rb]
        pzb = pzs[rb]

        def ch_body(ci, carry):
            cnta, cntb = carry
            off = ci * _L
            vx = p2x[pl.ds(off, _L)]
            vy = p2y[pl.ds(off, _L)]
            vz = p2z[pl.ds(off, _L)]
            jvec = (off + iota) & (T - 1)
            dxa = vx - pxa
            dya = vy - pya
            dza = vz - pza
            d2a = dxa * dxa + dya * dya + dza * dza
            ma = (d2a <= _R2) & (d2a > 1e-4)
            dxb = vx - pxb
            dyb = vy - pyb
            dzb = vz - pzb
            d2b = dxb * dxb + dyb * dyb + dzb * dzb
            mb = (d2b <= _R2) & (d2b > 1e-4)
            plsc.store_compressed(cd2.at[pl.ds(cnta, _L)], d2a, mask=ma)
            plsc.store_compressed(ck.at[pl.ds(cnta, _L)], jvec, mask=ma)
            plsc.store_compressed(cd2b.at[pl.ds(cntb, _L)], d2b, mask=mb)
            plsc.store_compressed(ckb.at[pl.ds(cntb, _L)], jvec, mask=mb)
            return (cnta + jnp.sum(ma.astype(jnp.int32)),
                    cntb + jnp.sum(mb.astype(jnp.int32)))

        cnta, cntb = lax.fori_loop(0, nch_scan, ch_body,
                                   (jnp.int32(0), jnp.int32(0)))
        cd2[pl.ds(cnta, _L)] = inf16
        cd2b[pl.ds(cntb, _L)] = inf16

        nch_ca = (cnta + _L - 1) // _L
        nch_cb = (cntb + _L - 1) // _L
        nch_mx = jnp.maximum(nch_ca, nch_cb)

        def cm_body(ci, _):
            mva = jnp.min(cd2[pl.ds(ci * _L, _L)])
            mvb = jnp.min(cd2b[pl.ds(ci * _L, _L)])
            mva = jnp.where(ci < nch_ca, mva, jnp.inf)
            mvb = jnp.where(ci < nch_cb, mvb, jnp.inf)
            plsc.store_compressed(cmin.at[pl.ds(ci, _L)],
                                  jnp.full((_L,), mva, jnp.float32), mask=lane0)
            plsc.store_compressed(cminb.at[pl.ds(ci, _L)],
                                  jnp.full((_L,), mvb, jnp.float32), mask=lane0)
            return 0

        lax.fori_loop(0, nch_mx, cm_body, 0)
        cmin[pl.ds(nch_mx, _L)] = inf16
        cminb[pl.ds(nch_mx, _L)] = inf16
        ncm = (nch_mx + _L - 1) // _L

        def ex_body(n, carry):
            kacca, dacca, kaccb, daccb = carry

            def gm_body(ci, gcarry):
                bva, bia, bvb, bib = gcarry
                va = cmin[pl.ds(ci * _L, _L)]
                vb = cminb[pl.ds(ci * _L, _L)]
                mva = jnp.min(va)
                mvb = jnp.min(vb)
                lanea = jnp.min(plsc.all_reduce_ffs(va == mva))
                laneb = jnp.min(plsc.all_reduce_ffs(vb == mvb))
                bettera = mva < bva
                betterb = mvb < bvb
                return (jnp.where(bettera, mva, bva),
                        jnp.where(bettera, ci * _L + lanea, bia),
                        jnp.where(betterb, mvb, bvb),
                        jnp.where(betterb, ci * _L + laneb, bib))

            gmina, fcha, gminb, fchb = lax.fori_loop(
                0, ncm, gm_body,
                (jnp.float32(jnp.inf), jnp.int32(0),
                 jnp.float32(jnp.inf), jnp.int32(0)))
            coffa = fcha * _L
            coffb = fchb * _L
            va = cd2[pl.ds(coffa, _L)]
            vb = cd2b[pl.ds(coffb, _L)]
            lanea = jnp.min(plsc.all_reduce_ffs(va == gmina))
            laneb = jnp.min(plsc.all_reduce_ffs(vb == gminb))
            vka = ck[pl.ds(coffa, _L)]
            vkb = ckb[pl.ds(coffb, _L)]
            kvala = jnp.min(jnp.where(iota == lanea, vka, jnp.int32(2 ** 30)))
            kvalb = jnp.min(jnp.where(iota == laneb, vkb, jnp.int32(2 ** 30)))
            sel = iota == (n % _L)
            kacca = jnp.where(sel, jnp.full((_L,), kvala, jnp.int32), kacca)
            dacca = jnp.where(sel, jnp.full((_L,), gmina, jnp.float32), dacca)
            kaccb = jnp.where(sel, jnp.full((_L,), kvalb, jnp.int32), kaccb)
            daccb = jnp.where(sel, jnp.full((_L,), gminb, jnp.float32), daccb)
            v2a = jnp.where(iota == lanea, jnp.inf, va)
            v2b = jnp.where(iota == laneb, jnp.inf, vb)
            cd2[pl.ds(coffa, _L)] = v2a
            cd2b[pl.ds(coffb, _L)] = v2b
            plsc.store_compressed(cmin.at[pl.ds(fcha, _L)],
                                  jnp.full((_L,), jnp.min(v2a), jnp.float32),
                                  mask=lane0)
            plsc.store_compressed(cminb.at[pl.ds(fchb, _L)],
                                  jnp.full((_L,), jnp.min(v2b), jnp.float32),
                                  mask=lane0)
            return kacca, dacca, kaccb, daccb

        lrow = base - b * T + ra
        a_ia = plsc.load_gather(atb_vm, [jnp.full((_L,), lrow, jnp.int32)])
        a_ib = plsc.load_gather(atb_vm, [jnp.full((_L,), lrow + 1, jnp.int32)])

        def emit_half(h0, r, a_i, kacc, dacc):
            outk_v[r, pl.ds(h0, _L)] = kacc
            outd_v[r, pl.ds(h0, _L)] = dacc
            a_j = plsc.load_gather(atb_vm, [kacc & (T - 1)])
            etype = a_i * NUM_TYPES + a_j
            outm_v[r, pl.ds(h0, _L)] = plsc.load_gather(mul_vm, [etype])
            outb_v[r, pl.ds(h0, _L)] = plsc.load_gather(bias_vm, [etype])

        z16i = jnp.zeros((_L,), jnp.int32)
        z16f = jnp.zeros((_L,), jnp.float32)
        acc = lax.fori_loop(0, _L, ex_body, (z16i, z16f, z16i, z16f))
        emit_half(0, ra, a_ia, acc[0], acc[1])
        emit_half(0, rb, a_ib, acc[2], acc[3])
        acc = lax.fori_loop(_L, MAX_NBR, ex_body, (z16i, z16f, z16i, z16f))
        emit_half(_L, ra, a_ia, acc[0], acc[1])
        emit_half(_L, rb, a_ib, acc[2], acc[3])
        return 0

    lax.fori_loop(0, _RPW // 2, pair_body, 0)
    pltpu.sync_copy(outk_v, outk_hbm.at[pl.ds(base, _RPW)])
    pltpu.sync_copy(outd_v, outd_hbm.at[pl.ds(base, _RPW)])
    pltpu.sync_copy(outm_v, outm_hbm.at[pl.ds(base, _RPW)])
    pltpu.sync_copy(outb_v, outb_hbm.at[pl.ds(base, _RPW)])


_sc_select = functools.partial(
    pl.kernel,
    out_type=[jax.ShapeDtypeStruct((B * T, MAX_NBR), jnp.int32),
              jax.ShapeDtypeStruct((B * T, MAX_NBR), jnp.float32),
              jax.ShapeDtypeStruct((B * T, MAX_NBR), jnp.float32),
              jax.ShapeDtypeStruct((B * T, MAX_NBR), jnp.float32)],
    mesh=plsc.VectorSubcoreMesh(core_axis_name="c", subcore_axis_name="s"),
    scratch_types=[
        pltpu.VMEM((NCAND,), jnp.float32),
        pltpu.VMEM((NCAND,), jnp.float32),
        pltpu.VMEM((NCAND,), jnp.float32),
        pltpu.VMEM((_RPW, _L), jnp.float32),
        pltpu.VMEM((_RPW, _L), jnp.float32),
        pltpu.VMEM((_RPW, _L), jnp.float32),
        pltpu.VMEM((_L,), jnp.int32),
        pltpu.VMEM((T,), jnp.int32),
        pltpu.VMEM((NUM_TYPES * NUM_TYPES,), jnp.float32),
        pltpu.VMEM((NUM_TYPES * NUM_TYPES,), jnp.float32),
        pltpu.VMEM((NCAND + _L,), jnp.float32),
        pltpu.VMEM((NCAND + _L,), jnp.int32),
        pltpu.VMEM((_NCH + _L,), jnp.float32),
        pltpu.VMEM((NCAND + _L,), jnp.float32),
        pltpu.VMEM((NCAND + _L,), jnp.int32),
        pltpu.VMEM((_NCH + _L,), jnp.float32),
        pltpu.VMEM((_RPW, MAX_NBR), jnp.int32),
        pltpu.VMEM((_RPW, MAX_NBR), jnp.float32),
        pltpu.VMEM((_RPW, MAX_NBR), jnp.float32),
        pltpu.VMEM((_RPW, MAX_NBR), jnp.float32),
    ],
    compiler_params=pltpu.CompilerParams(needs_layout_passes=False),
)(_sc_select_body)


def _select_body(p2x_ref, p2y_ref, p2z_ref, px_ref, py_ref, pz_ref,
                 idx_ref, d2_ref):
    p2x = p2x_ref[0, 0, :]
    p2y = p2y_ref[0, 0, :]
    p2z = p2z_ref[0, 0, :]
    px = px_ref[0]
    py = py_ref[0]
    pz = pz_ref[0]
    dx = p2x - px
    dy = p2y - py
    dz = p2z - pz
    d2 = dx * dx + dy * dy + dz * dz
    ok = (d2 <= RADIUS * RADIUS) & (d2 > 1e-4)
    d2m = jnp.where(ok, d2, jnp.inf)
    iota = jax.lax.broadcasted_iota(jnp.int32, (_RG, NCAND), 1)
    idx_cols = []
    d2_cols = []
    for _ in range(MAX_NBR):
        rowmin = jnp.min(d2m, axis=1, keepdims=True)
        cand = jnp.where(d2m == rowmin, iota, jnp.int32(2 ** 30))
        amin = jnp.min(cand, axis=1, keepdims=True)
        idx_cols.append(amin)
        d2_cols.append(rowmin)
        d2m = jnp.where(iota == amin, jnp.inf, d2m)
    idx_ref[0] = jnp.concatenate(idx_cols, axis=1)
    d2_ref[0] = jnp.concatenate(d2_cols, axis=1)


def _edge_body(idx_ref, d2_ref, mule_ref, biase_ref,
               means_ref, stds_ref, w1_ref, b1_ref, w2_ref, b2_ref, out_ref):
    k = idx_ref[0]
    d2s = d2_ref[0]
    valid = d2s <= RADIUS * RADIUS
    j = k.astype(jnp.float32)

    iota_t = jax.lax.broadcasted_iota(jnp.int32, (_E, T), 1).astype(jnp.float32)
    oj = (j == iota_t).astype(jnp.float32)
    mul_e = mule_ref[0]
    bias_e = biase_ref[0]

    d2c = jnp.where(valid, d2s, 0.0)
    dist = jnp.sqrt(d2c + 1e-12)
    x = mul_e * dist + bias_e

    std = jnp.abs(stds_ref[...]) + 1e-5
    pre = (x - means_ref[...]) / std
    g = jnp.exp(-0.5 * pre * pre) / (_SQRT_2PI * std)
    h1 = jax.nn.gelu(jnp.dot(g, w1_ref[...], preferred_element_type=jnp.float32, precision=jax.lax.Precision.HIGHEST)
                     + b1_ref[...])
    h = jnp.dot(h1, w2_ref[...], preferred_element_type=jnp.float32, precision=jax.lax.Precision.HIGHEST) + b2_ref[...]
    h = jnp.where(valid, h, 0.0)
    hb = h.astype(jnp.bfloat16).astype(jnp.float32)
    hr = h - hb
    htb = hb.T
    htr = hr.T

    for i in range(_BI):
        sl = slice(i * MAX_NBR, (i + 1) * MAX_NBR)
        si = oj[sl, :]
        out_ref[0, :, i, :] = (
            jnp.dot(htb[:, sl], si, preferred_element_type=jnp.float32)
            + jnp.dot(htr[:, sl], si, preferred_element_type=jnp.float32))


def kernel(pos, atom_types, natoms, cell, means, stds, mul_w, bias_w,
           W1, b1, W2, b2):
    del natoms
    cross_a2a3 = jnp.cross(cell[:, 1], cell[:, 2])
    cell_vol = jnp.sum(cell[:, 0] * cross_a2a3, axis=-1, keepdims=True)
    crosses = [cross_a2a3, jnp.cross(cell[:, 2], cell[:, 0]),
               jnp.cross(cell[:, 0], cell[:, 1])]
    pbc = (True, True, False)
    rep_caps = []
    for dim in range(3):
        if pbc[dim]:
            inv = jnp.linalg.norm(crosses[dim] / cell_vol, axis=-1)
            rep_caps.append(jnp.minimum(jnp.max(jnp.ceil(RADIUS * inv)), float(MAX_REP)))
        else:
            rep_caps.append(jnp.asarray(0.0, jnp.float32))
    static_reps = [MAX_REP if pbc[dim] else 0 for dim in range(3)]
    cells_per_dim = [np.arange(-r, r + 1, dtype=np.float32) for r in static_reps]
    mg = np.meshgrid(*cells_per_dim, indexing="ij")
    unit_cell = jnp.asarray(np.stack([m.reshape(-1) for m in mg], axis=1))
    cell_ok = jnp.ones((C,), bool)
    for dim in range(3):
        cell_ok = cell_ok & (jnp.abs(unit_cell[:, dim]) <= rep_caps[dim])
    pbc_off = jnp.einsum("ci,bij->bcj", unit_cell, cell)
    order = jnp.argsort(jnp.logical_not(cell_ok).astype(jnp.int32), stable=True)
    n_alive = jnp.sum(cell_ok.astype(jnp.int32))
    pbc_off = pbc_off[:, order, :]
    pos2 = pbc_off[:, :, None, :] + pos[:, None, :, :]
    pos2 = pos2.reshape(B, NCAND, 3)

    p2x = pos2[..., 0]
    p2y = pos2[..., 1]
    p2z = pos2[..., 2]
    pos_flat = pos.reshape(B * T, 3)
    pxs = jnp.broadcast_to(pos_flat[:, 0:1], (B * T, _L))
    pys = jnp.broadcast_to(pos_flat[:, 1:2], (B * T, _L))
    pzs = jnp.broadcast_to(pos_flat[:, 2:3], (B * T, _L))
    nav = jnp.full((_L,), n_alive, jnp.int32)

    sel_idx, sel_d2, mul_e, bias_e = _sc_select(
        p2x, p2y, p2z, pxs, pys, pzs, nav, atom_types.astype(jnp.int32),
        mul_w, bias_w)
    nsteps = B * T // _BI
    steps_per_b = T // _BI
    sel_idx = sel_idx.reshape(nsteps, _E, 1)
    sel_d2 = sel_d2.reshape(nsteps, _E, 1)
    mul_e = mul_e.reshape(nsteps, _E, 1)
    bias_e = bias_e.reshape(nsteps, _E, 1)

    out = pl.pallas_call(
        _edge_body,
        grid=(nsteps,),
        in_specs=[
            pl.BlockSpec((1, _E, 1), lambda g: (g, 0, 0)),
            pl.BlockSpec((1, _E, 1), lambda g: (g, 0, 0)),
            pl.BlockSpec((1, _E, 1), lambda g: (g, 0, 0)),
            pl.BlockSpec((1, _E, 1), lambda g: (g, 0, 0)),
            pl.BlockSpec((1, K), lambda g: (0, 0)),
            pl.BlockSpec((1, K), lambda g: (0, 0)),
            pl.BlockSpec((K, K), lambda g: (0, 0)),
            pl.BlockSpec((1, K), lambda g: (0, 0)),
            pl.BlockSpec((K, H), lambda g: (0, 0)),
            pl.BlockSpec((1, H), lambda g: (0, 0)),
        ],
        out_specs=pl.BlockSpec(
            (1, H, _BI, T), lambda g: (g // steps_per_b, 0, g % steps_per_b, 0)),
        out_shape=jax.ShapeDtypeStruct((B, H, T, T), jnp.float32),
    )(sel_idx, sel_d2, mul_e, bias_e,
      means.reshape(1, K), stds.reshape(1, K), W1, b1.reshape(1, K),
      W2, b2.reshape(1, H))
    return out
